# Initial kernel scaffold; baseline (speedup 1.0000x reference)
import math

import jax
import jax.numpy as jnp
from jax import lax
from jax.experimental import pallas as pl
from jax.experimental.pallas import tpu as pltpu
from jax.experimental.pallas import tpu_sc as plsc

N = 50000
H = 64
HH = H // 2
NC, NS = 2, 16
LANES = 128
KA = 14
E = 800000
EROWS = 6272
EPAD = EROWS * LANES
DEG_TROWS = EROWS // (NC * NS)
AGG_TROWS = EROWS // NS
ZROWS = 3128
ACC_ROWS = ZROWS * NS
TRASH = N
BR = 2000
GRID = N // BR

_SQRT2 = math.sqrt(2.0)


def _gelu(x):
    return 0.5 * x * (1.0 + lax.erf(x / _SQRT2))


def _deg_body(colh, zeros1, ones_hbm, out0, out1, acc, cidx, ones_v, bounce):
    c = lax.axis_index("c")
    s = lax.axis_index("s")
    wid = c * NS + s
    pltpu.sync_copy(zeros1, acc.at[pl.ds(s * ZROWS, ZROWS)])
    pltpu.sync_copy(ones_hbm, ones_v)
    plsc.subcore_barrier()

    base = wid * DEG_TROWS

    def chunk(i, carry):
        off = base + i * KA
        pltpu.sync_copy(colh.at[pl.ds(off, KA)], cidx)
        for j in range(KA):
            pltpu.sync_copy(ones_v, acc.at[cidx.at[j]], add=True)
        return carry

    lax.fori_loop(0, DEG_TROWS // KA, chunk, 0)
    plsc.subcore_barrier()

    @pl.when(c == 0)
    def _():
        pltpu.sync_copy(acc.at[pl.ds(s * ZROWS, ZROWS)], bounce)
        pltpu.sync_copy(bounce, out0.at[pl.ds(s * ZROWS, ZROWS)])

    @pl.when(c == 1)
    def _():
        pltpu.sync_copy(acc.at[pl.ds(s * ZROWS, ZROWS)], bounce)
        pltpu.sync_copy(bounce, out1.at[pl.ds(s * ZROWS, ZROWS)])


_deg_call = pl.kernel(
    _deg_body,
    out_type=[jax.ShapeDtypeStruct((ACC_ROWS,), jnp.float32)] * 2,
    mesh=plsc.VectorSubcoreMesh(
        core_axis_name="c", subcore_axis_name="s", num_cores=NC, num_subcores=NS
    ),
    scratch_types=[
        pltpu.VMEM_SHARED((ACC_ROWS,), jnp.float32),
        pltpu.VMEM((KA, LANES), jnp.int32),
        pltpu.VMEM((LANES,), jnp.float32),
        pltpu.VMEM((ZROWS,), jnp.float32),
    ],
)


def _agg_pipe(xs_hbm, out_hbm, rowh, colh, zeros32, acc, ridx, cidx, rows_v,
              sem, bounce, s):
    pltpu.sync_copy(zeros32, acc.at[pl.ds(s * ZROWS, ZROWS)])
    plsc.subcore_barrier()

    base = s * AGG_TROWS

    def chunk(i, carry):
        off = base + i * KA
        pltpu.sync_copy(rowh.at[pl.ds(off, KA)], ridx)
        pltpu.sync_copy(colh.at[pl.ds(off, KA)], cidx)
        cps = [
            pltpu.async_copy(xs_hbm.at[ridx.at[j]], rows_v.at[j], sem)
            for j in range(KA)
        ]
        for cp in cps:
            cp.wait()
        for j in range(KA):
            pltpu.sync_copy(rows_v.at[j], acc.at[cidx.at[j]], add=True)
        return carry

    lax.fori_loop(0, AGG_TROWS // KA, chunk, 0)
    plsc.subcore_barrier()

    half = ZROWS // 2
    for p in range(2):
        r0 = s * ZROWS + p * half
        pltpu.sync_copy(acc.at[pl.ds(r0, half)], bounce)
        pltpu.sync_copy(bounce, out_hbm.at[pl.ds(r0, half)])


def _agg_body(xs0, xs1, rowh, colh, zeros32, out0, out1, acc, ridx, cidx,
              rows_v, sem, bounce):
    c = lax.axis_index("c")
    s = lax.axis_index("s")

    @pl.when(c == 0)
    def _():
        _agg_pipe(xs0, out0, rowh, colh, zeros32, acc, ridx, cidx, rows_v,
                  sem, bounce, s)

    @pl.when(c == 1)
    def _():
        _agg_pipe(xs1, out1, rowh, colh, zeros32, acc, ridx, cidx, rows_v,
                  sem, bounce, s)


_agg_call = pl.kernel(
    _agg_body,
    out_type=[jax.ShapeDtypeStruct((ACC_ROWS, HH), jnp.float32)] * 2,
    mesh=plsc.VectorSubcoreMesh(
        core_axis_name="c", subcore_axis_name="s", num_cores=NC, num_subcores=NS
    ),
    scratch_types=[
        pltpu.VMEM_SHARED((ACC_ROWS, HH), jnp.float32),
        pltpu.VMEM((KA, LANES), jnp.int32),
        pltpu.VMEM((KA, LANES), jnp.int32),
        pltpu.VMEM((KA, LANES, HH), jnp.float32),
        pltpu.SemaphoreType.DMA,
        pltpu.VMEM((ZROWS // 2, HH), jnp.float32),
    ],
)


def _k1_body(x, W_in, b_in, Wc0, p0, p1, h_o, dis_o, xs0_o, xs1_o):
    dis = lax.rsqrt(p0[...] + p1[...] + 1.0)
    h = jnp.dot(x[...], W_in[...], preferred_element_type=jnp.float32) + b_in[...]
    xs = jnp.dot(h, Wc0[...], preferred_element_type=jnp.float32) * dis
    h_o[...] = h
    dis_o[...] = dis
    xs0_o[...] = xs[:, :HH]
    xs1_o[...] = xs[:, HH:]


def _k2_body(a0, a1, xs0, xs1, h_prev, dis, bcp, Wc, h_o, xs0_o, xs1_o):
    agg = jnp.concatenate([a0[...] + xs0[...], a1[...] + xs1[...]], axis=1)
    out = agg * dis[...] + bcp[...]
    h = _gelu(out) + h_prev[...]
    xs = jnp.dot(h, Wc[...], preferred_element_type=jnp.float32) * dis[...]
    h_o[...] = h
    xs0_o[...] = xs[:, :HH]
    xs1_o[...] = xs[:, HH:]


def _k3_body(a0, a1, xs0, xs1, h_prev, dis, bcp, g, b, W1, b1, W2, b2, y_o):
    agg = jnp.concatenate([a0[...] + xs0[...], a1[...] + xs1[...]], axis=1)
    out = agg * dis[...] + bcp[...]
    h = _gelu(out) + h_prev[...]
    mu = jnp.mean(h, axis=-1, keepdims=True)
    var = jnp.mean((h - mu) ** 2, axis=-1, keepdims=True)
    hn = (h - mu) * lax.rsqrt(var + 1e-5) * g[...] + b[...]
    h1 = _gelu(jnp.dot(hn, W1[...], preferred_element_type=jnp.float32) + b1[...])
    y_o[...] = jnp.dot(h1, W2[...], preferred_element_type=jnp.float32) + b2[...]


def _row_spec(cols):
    return pl.BlockSpec((BR, cols), lambda i: (i, 0))


def _full_spec(r, c):
    return pl.BlockSpec((r, c), lambda i: (0, 0))


_k1 = pl.pallas_call(
    _k1_body,
    grid=(GRID,),
    in_specs=[
        _row_spec(128), _full_spec(128, H), _full_spec(1, H), _full_spec(H, H),
        _row_spec(1), _row_spec(1),
    ],
    out_specs=[_row_spec(H), _row_spec(1), _row_spec(HH), _row_spec(HH)],
    out_shape=[
        jax.ShapeDtypeStruct((N, H), jnp.float32),
        jax.ShapeDtypeStruct((N, 1), jnp.float32),
        jax.ShapeDtypeStruct((N, HH), jnp.float32),
        jax.ShapeDtypeStruct((N, HH), jnp.float32),
    ],
)

_k2 = pl.pallas_call(
    _k2_body,
    grid=(GRID,),
    in_specs=[
        _row_spec(HH), _row_spec(HH), _row_spec(HH), _row_spec(HH),
        _row_spec(H), _row_spec(1), _full_spec(1, H), _full_spec(H, H),
    ],
    out_specs=[_row_spec(H), _row_spec(HH), _row_spec(HH)],
    out_shape=[
        jax.ShapeDtypeStruct((N, H), jnp.float32),
        jax.ShapeDtypeStruct((N, HH), jnp.float32),
        jax.ShapeDtypeStruct((N, HH), jnp.float32),
    ],
)

_k3 = pl.pallas_call(
    _k3_body,
    grid=(GRID,),
    in_specs=[
        _row_spec(HH), _row_spec(HH), _row_spec(HH), _row_spec(HH),
        _row_spec(H), _row_spec(1), _full_spec(1, H), _full_spec(1, H),
        _full_spec(1, H), _full_spec(H, H), _full_spec(1, H), _full_spec(H, 1),
        _full_spec(1, 1),
    ],
    out_specs=[_row_spec(1)],
    out_shape=[jax.ShapeDtypeStruct((N, 1), jnp.float32)],
)


def kernel(x, edge_index, W_in, b_in, Wc, bc, ln_g, ln_b, W_h1, b_h1, W_h2, b_h2):
    ei = edge_index.astype(jnp.int32)
    pad = EPAD - E
    row = jnp.concatenate([ei[0], jnp.zeros((pad,), jnp.int32)])
    col = jnp.concatenate([ei[1], jnp.full((pad,), TRASH, jnp.int32)])
    rowh = row.reshape(EROWS, LANES)
    colh = col.reshape(EROWS, LANES)
    zeros32 = jnp.zeros((ZROWS, HH), jnp.float32)
    zeros1 = jnp.zeros((ZROWS,), jnp.float32)
    ones128 = jnp.ones((LANES,), jnp.float32)

    p0, p1 = _deg_call(colh, zeros1, ones128)
    h, dis, xs0, xs1 = _k1(
        x, W_in, b_in.reshape(1, H), Wc[0],
        p0.reshape(ACC_ROWS, 1), p1.reshape(ACC_ROWS, 1),
    )
    for i in range(Wc.shape[0]):
        a0, a1 = _agg_call(xs0, xs1, rowh, colh, zeros32)
        if i + 1 < Wc.shape[0]:
            h, xs0, xs1 = _k2(a0, a1, xs0, xs1, h, dis,
                              bc[i].reshape(1, H), Wc[i + 1])
        else:
            (y,) = _k3(a0, a1, xs0, xs1, h, dis, bc[i].reshape(1, H),
                       ln_g.reshape(1, H), ln_b.reshape(1, H),
                       W_h1, b_h1.reshape(1, H), W_h2, b_h2.reshape(1, 1))
    return y


# speedup vs baseline: 11.8199x; 11.8199x over previous
import math

import jax
import jax.numpy as jnp
from jax import lax
from jax.experimental import pallas as pl
from jax.experimental.pallas import tpu as pltpu
from jax.experimental.pallas import tpu_sc as plsc

N = 50000
H = 64
HH = H // 2
NC, NS = 2, 16
LANES = 128
KA = 4
BOUNCE = KA * 128
E = 800000
EROWS = 6400
EPAD = EROWS * LANES
DEG_TROWS = EROWS // (NC * NS)
AGG_TROWS = EROWS // NS
ZROWS = 3128
ACC_ROWS = ZROWS * NS
TRASH = N
BR = 2000
GRID = N // BR

_SQRT2 = math.sqrt(2.0)


def _gelu(x):
    return 0.5 * x * (1.0 + lax.erf(x / _SQRT2))


def _deg_body(colh, zeros1, ones_hbm, out0, out1, acc, cidx, ones_v, bounce):
    c = lax.axis_index("c")
    s = lax.axis_index("s")
    wid = c * NS + s
    pltpu.sync_copy(zeros1, bounce)
    pltpu.sync_copy(bounce, acc.at[pl.ds(s * ZROWS, ZROWS)])
    pltpu.sync_copy(ones_hbm, ones_v)
    plsc.subcore_barrier()

    base = wid * DEG_TROWS

    def chunk(i, carry):
        off = base + i * KA
        pltpu.sync_copy(colh.at[pl.ds(off, KA)], cidx)
        for j in range(KA):
            pltpu.sync_copy(ones_v, acc.at[cidx.at[j]], add=True)
        return carry

    lax.fori_loop(0, DEG_TROWS // KA, chunk, 0)
    plsc.subcore_barrier()

    @pl.when(c == 0)
    def _():
        pltpu.sync_copy(acc.at[pl.ds(s * ZROWS, ZROWS)], bounce)
        pltpu.sync_copy(bounce, out0.at[pl.ds(s * ZROWS, ZROWS)])

    @pl.when(c == 1)
    def _():
        pltpu.sync_copy(acc.at[pl.ds(s * ZROWS, ZROWS)], bounce)
        pltpu.sync_copy(bounce, out1.at[pl.ds(s * ZROWS, ZROWS)])


_deg_call = pl.kernel(
    _deg_body,
    out_type=[jax.ShapeDtypeStruct((ACC_ROWS,), jnp.float32)] * 2,
    mesh=plsc.VectorSubcoreMesh(
        core_axis_name="c", subcore_axis_name="s", num_cores=NC, num_subcores=NS
    ),
    scratch_types=[
        pltpu.VMEM_SHARED((ACC_ROWS,), jnp.float32),
        pltpu.VMEM((KA, LANES), jnp.int32),
        pltpu.VMEM((LANES,), jnp.float32),
        pltpu.VMEM((ZROWS,), jnp.float32),
    ],
)


def _agg_pipe(xs_hbm, out_hbm, rowh, colh, zeros32, acc, ridx, cidx, rows_v,
              sem, s):
    pltpu.sync_copy(zeros32, rows_v)
    for q in range(ZROWS // BOUNCE):
        pltpu.sync_copy(rows_v, acc.at[pl.ds(s * ZROWS + q * BOUNCE, BOUNCE)])
    tail0 = (ZROWS // BOUNCE) * BOUNCE
    pltpu.sync_copy(rows_v.at[pl.ds(0, ZROWS - tail0)],
                    acc.at[pl.ds(s * ZROWS + tail0, ZROWS - tail0)])
    plsc.subcore_barrier()

    base = s * AGG_TROWS

    def chunk(i, carry):
        off = base + i * KA
        pltpu.sync_copy(rowh.at[pl.ds(off, KA)], ridx)
        pltpu.sync_copy(colh.at[pl.ds(off, KA)], cidx)
        cps = [
            pltpu.async_copy(xs_hbm.at[ridx.at[j]],
                             rows_v.at[pl.ds(j * LANES, LANES)], sem)
            for j in range(KA)
        ]
        for cp in cps:
            cp.wait()
        for j in range(KA):
            pltpu.sync_copy(rows_v.at[pl.ds(j * LANES, LANES)],
                            acc.at[cidx.at[j]], add=True)
        return carry

    lax.fori_loop(0, AGG_TROWS // KA, chunk, 0)
    plsc.subcore_barrier()

    for q in range(ZROWS // BOUNCE):
        r0 = s * ZROWS + q * BOUNCE
        pltpu.sync_copy(acc.at[pl.ds(r0, BOUNCE)], rows_v)
        pltpu.sync_copy(rows_v, out_hbm.at[pl.ds(r0, BOUNCE)])
    r0 = s * ZROWS + tail0
    pltpu.sync_copy(acc.at[pl.ds(r0, ZROWS - tail0)],
                    rows_v.at[pl.ds(0, ZROWS - tail0)])
    pltpu.sync_copy(rows_v.at[pl.ds(0, ZROWS - tail0)],
                    out_hbm.at[pl.ds(r0, ZROWS - tail0)])


def _agg_body(xs0, xs1, rowh, colh, zeros32, out0, out1, acc, ridx, cidx,
              rows_v, sem):
    c = lax.axis_index("c")
    s = lax.axis_index("s")

    @pl.when(c == 0)
    def _():
        _agg_pipe(xs0, out0, rowh, colh, zeros32, acc, ridx, cidx, rows_v,
                  sem, s)

    @pl.when(c == 1)
    def _():
        _agg_pipe(xs1, out1, rowh, colh, zeros32, acc, ridx, cidx, rows_v,
                  sem, s)


_agg_call = pl.kernel(
    _agg_body,
    out_type=[jax.ShapeDtypeStruct((ACC_ROWS, HH), jnp.float32)] * 2,
    mesh=plsc.VectorSubcoreMesh(
        core_axis_name="c", subcore_axis_name="s", num_cores=NC, num_subcores=NS
    ),
    compiler_params=pltpu.CompilerParams(use_tc_tiling_on_sc=False),
    scratch_types=[
        pltpu.VMEM_SHARED((ACC_ROWS, HH), jnp.float32),
        pltpu.VMEM((KA, LANES), jnp.int32),
        pltpu.VMEM((KA, LANES), jnp.int32),
        pltpu.VMEM((BOUNCE, HH), jnp.float32),
        pltpu.SemaphoreType.DMA,
    ],
)


def _k1_body(x, W_in, b_in, Wc0, p0, p1, h_o, dis_o, xs0_o, xs1_o):
    dis = lax.rsqrt(p0[...] + p1[...] + 1.0)
    h = jnp.dot(x[...], W_in[...], preferred_element_type=jnp.float32) + b_in[...]
    xs = jnp.dot(h, Wc0[...], preferred_element_type=jnp.float32) * dis
    h_o[...] = h
    dis_o[...] = dis
    xs0_o[...] = xs[:, :HH]
    xs1_o[...] = xs[:, HH:]


def _k2_body(a0, a1, xs0, xs1, h_prev, dis, bcp, Wc, h_o, xs0_o, xs1_o):
    agg = jnp.concatenate([a0[...] + xs0[...], a1[...] + xs1[...]], axis=1)
    out = agg * dis[...] + bcp[...]
    h = _gelu(out) + h_prev[...]
    xs = jnp.dot(h, Wc[...], preferred_element_type=jnp.float32) * dis[...]
    h_o[...] = h
    xs0_o[...] = xs[:, :HH]
    xs1_o[...] = xs[:, HH:]


def _k3_body(a0, a1, xs0, xs1, h_prev, dis, bcp, g, b, W1, b1, W2, b2, y_o):
    agg = jnp.concatenate([a0[...] + xs0[...], a1[...] + xs1[...]], axis=1)
    out = agg * dis[...] + bcp[...]
    h = _gelu(out) + h_prev[...]
    mu = jnp.mean(h, axis=-1, keepdims=True)
    var = jnp.mean((h - mu) ** 2, axis=-1, keepdims=True)
    hn = (h - mu) * lax.rsqrt(var + 1e-5) * g[...] + b[...]
    h1 = _gelu(jnp.dot(hn, W1[...], preferred_element_type=jnp.float32) + b1[...])
    y_o[...] = jnp.dot(h1, W2[...], preferred_element_type=jnp.float32) + b2[...]


def _row_spec(cols):
    return pl.BlockSpec((BR, cols), lambda i: (i, 0))


def _full_spec(r, c):
    return pl.BlockSpec((r, c), lambda i: (0, 0))


_k1 = pl.pallas_call(
    _k1_body,
    grid=(GRID,),
    in_specs=[
        _row_spec(128), _full_spec(128, H), _full_spec(1, H), _full_spec(H, H),
        _row_spec(1), _row_spec(1),
    ],
    out_specs=[_row_spec(H), _row_spec(1), _row_spec(HH), _row_spec(HH)],
    out_shape=[
        jax.ShapeDtypeStruct((N, H), jnp.float32),
        jax.ShapeDtypeStruct((N, 1), jnp.float32),
        jax.ShapeDtypeStruct((N, HH), jnp.float32),
        jax.ShapeDtypeStruct((N, HH), jnp.float32),
    ],
)

_k2 = pl.pallas_call(
    _k2_body,
    grid=(GRID,),
    in_specs=[
        _row_spec(HH), _row_spec(HH), _row_spec(HH), _row_spec(HH),
        _row_spec(H), _row_spec(1), _full_spec(1, H), _full_spec(H, H),
    ],
    out_specs=[_row_spec(H), _row_spec(HH), _row_spec(HH)],
    out_shape=[
        jax.ShapeDtypeStruct((N, H), jnp.float32),
        jax.ShapeDtypeStruct((N, HH), jnp.float32),
        jax.ShapeDtypeStruct((N, HH), jnp.float32),
    ],
)

_k3 = pl.pallas_call(
    _k3_body,
    grid=(GRID,),
    in_specs=[
        _row_spec(HH), _row_spec(HH), _row_spec(HH), _row_spec(HH),
        _row_spec(H), _row_spec(1), _full_spec(1, H), _full_spec(1, H),
        _full_spec(1, H), _full_spec(H, H), _full_spec(1, H), _full_spec(H, 1),
        _full_spec(1, 1),
    ],
    out_specs=[_row_spec(1)],
    out_shape=[jax.ShapeDtypeStruct((N, 1), jnp.float32)],
)


def kernel(x, edge_index, W_in, b_in, Wc, bc, ln_g, ln_b, W_h1, b_h1, W_h2, b_h2):
    ei = edge_index.astype(jnp.int32)
    pad = EPAD - E
    row = jnp.concatenate([ei[0], jnp.zeros((pad,), jnp.int32)])
    col = jnp.concatenate([ei[1], jnp.full((pad,), TRASH, jnp.int32)])
    rowh = row.reshape(EROWS, LANES)
    colh = col.reshape(EROWS, LANES)
    zeros32 = jnp.zeros((BOUNCE, HH), jnp.float32)
    zeros1 = jnp.zeros((ZROWS,), jnp.float32)
    ones128 = jnp.ones((LANES,), jnp.float32)

    p0, p1 = _deg_call(colh, zeros1, ones128)
    h, dis, xs0, xs1 = _k1(
        x, W_in, b_in.reshape(1, H), Wc[0],
        p0.reshape(ACC_ROWS, 1), p1.reshape(ACC_ROWS, 1),
    )
    for i in range(Wc.shape[0]):
        a0, a1 = _agg_call(xs0, xs1, rowh, colh, zeros32)
        if i + 1 < Wc.shape[0]:
            h, xs0, xs1 = _k2(a0, a1, xs0, xs1, h, dis,
                              bc[i].reshape(1, H), Wc[i + 1])
        else:
            (y,) = _k3(a0, a1, xs0, xs1, h, dis, bc[i].reshape(1, H),
                       ln_g.reshape(1, H), ln_b.reshape(1, H),
                       W_h1, b_h1.reshape(1, H), W_h2, b_h2.reshape(1, 1))
    return y


# speedup vs baseline: 12.1402x; 1.0271x over previous
import math

import jax
import jax.numpy as jnp
from jax import lax
from jax.experimental import pallas as pl
from jax.experimental.pallas import tpu as pltpu
from jax.experimental.pallas import tpu_sc as plsc

N = 50000
H = 64
HH = H // 2
NC, NS = 2, 16
LANES = 128
KA = 4
BOUNCE = KA * 128
E = 800000
EROWS = 6400
EPAD = EROWS * LANES
DEG_TROWS = EROWS // (NC * NS)
AGG_TROWS = EROWS // NS
ZROWS = 3128
ACC_ROWS = ZROWS * NS
TRASH = N
BR = 2000
GRID = N // BR

_SQRT2 = math.sqrt(2.0)


def _gelu(x):
    return 0.5 * x * (1.0 + lax.erf(x / _SQRT2))


def _deg_body(colh, zeros1, ones_hbm, out0, out1, acc, cidx, ones_v, bounce):
    c = lax.axis_index("c")
    s = lax.axis_index("s")
    wid = c * NS + s
    pltpu.sync_copy(zeros1, bounce)
    pltpu.sync_copy(bounce, acc.at[pl.ds(s * ZROWS, ZROWS)])
    pltpu.sync_copy(ones_hbm, ones_v)
    plsc.subcore_barrier()

    base = wid * DEG_TROWS

    def chunk(i, carry):
        off = base + i * KA
        pltpu.sync_copy(colh.at[pl.ds(off, KA)], cidx)
        for j in range(KA):
            pltpu.sync_copy(ones_v, acc.at[cidx.at[j]], add=True)
        return carry

    lax.fori_loop(0, DEG_TROWS // KA, chunk, 0)
    plsc.subcore_barrier()

    @pl.when(c == 0)
    def _():
        pltpu.sync_copy(acc.at[pl.ds(s * ZROWS, ZROWS)], bounce)
        pltpu.sync_copy(bounce, out0.at[pl.ds(s * ZROWS, ZROWS)])

    @pl.when(c == 1)
    def _():
        pltpu.sync_copy(acc.at[pl.ds(s * ZROWS, ZROWS)], bounce)
        pltpu.sync_copy(bounce, out1.at[pl.ds(s * ZROWS, ZROWS)])


_deg_call = pl.kernel(
    _deg_body,
    out_type=[jax.ShapeDtypeStruct((ACC_ROWS,), jnp.float32)] * 2,
    mesh=plsc.VectorSubcoreMesh(
        core_axis_name="c", subcore_axis_name="s", num_cores=NC, num_subcores=NS
    ),
    scratch_types=[
        pltpu.VMEM_SHARED((ACC_ROWS,), jnp.float32),
        pltpu.VMEM((KA, LANES), jnp.int32),
        pltpu.VMEM((LANES,), jnp.float32),
        pltpu.VMEM((ZROWS,), jnp.float32),
    ],
)


def _agg_pipe(xs_hbm, out_hbm, rowh, colh, zeros32, acc, ridx, cidx, rows_v,
              sem, s):
    pltpu.sync_copy(zeros32, rows_v)
    for q in range(ZROWS // BOUNCE):
        pltpu.sync_copy(rows_v, acc.at[pl.ds(s * ZROWS + q * BOUNCE, BOUNCE)])
    tail0 = (ZROWS // BOUNCE) * BOUNCE
    pltpu.sync_copy(rows_v.at[pl.ds(0, ZROWS - tail0)],
                    acc.at[pl.ds(s * ZROWS + tail0, ZROWS - tail0)])
    plsc.subcore_barrier()

    base = s * AGG_TROWS
    gsem0, gsem1, ssem0, ssem1 = sem
    half = KA // 2

    def chunk(i, carry):
        off = base + i * KA
        pltpu.sync_copy(rowh.at[pl.ds(off, KA)], ridx)
        pltpu.sync_copy(colh.at[pl.ds(off, KA)], cidx)
        g0 = [
            pltpu.async_copy(xs_hbm.at[ridx.at[j]],
                             rows_v.at[pl.ds(j * LANES, LANES)], gsem0)
            for j in range(half)
        ]
        g1 = [
            pltpu.async_copy(xs_hbm.at[ridx.at[j]],
                             rows_v.at[pl.ds(j * LANES, LANES)], gsem1)
            for j in range(half, KA)
        ]
        for cp in g0:
            cp.wait()
        s0 = [
            pltpu.async_copy(rows_v.at[pl.ds(j * LANES, LANES)],
                             acc.at[cidx.at[j]], ssem0, add=True)
            for j in range(half)
        ]
        for cp in g1:
            cp.wait()
        s1 = [
            pltpu.async_copy(rows_v.at[pl.ds(j * LANES, LANES)],
                             acc.at[cidx.at[j]], ssem1, add=True)
            for j in range(half, KA)
        ]
        for cp in s0:
            cp.wait()
        for cp in s1:
            cp.wait()
        return carry

    lax.fori_loop(0, AGG_TROWS // KA, chunk, 0)
    plsc.subcore_barrier()

    for q in range(ZROWS // BOUNCE):
        r0 = s * ZROWS + q * BOUNCE
        pltpu.sync_copy(acc.at[pl.ds(r0, BOUNCE)], rows_v)
        pltpu.sync_copy(rows_v, out_hbm.at[pl.ds(r0, BOUNCE)])
    r0 = s * ZROWS + tail0
    pltpu.sync_copy(acc.at[pl.ds(r0, ZROWS - tail0)],
                    rows_v.at[pl.ds(0, ZROWS - tail0)])
    pltpu.sync_copy(rows_v.at[pl.ds(0, ZROWS - tail0)],
                    out_hbm.at[pl.ds(r0, ZROWS - tail0)])


def _agg_body(xs0, xs1, rowh, colh, zeros32, out0, out1, acc, ridx, cidx,
              rows_v, sem):
    c = lax.axis_index("c")
    s = lax.axis_index("s")

    @pl.when(c == 0)
    def _():
        _agg_pipe(xs0, out0, rowh, colh, zeros32, acc, ridx, cidx, rows_v,
                  sem, s)

    @pl.when(c == 1)
    def _():
        _agg_pipe(xs1, out1, rowh, colh, zeros32, acc, ridx, cidx, rows_v,
                  sem, s)


_agg_call = pl.kernel(
    _agg_body,
    out_type=[jax.ShapeDtypeStruct((ACC_ROWS, HH), jnp.float32)] * 2,
    mesh=plsc.VectorSubcoreMesh(
        core_axis_name="c", subcore_axis_name="s", num_cores=NC, num_subcores=NS
    ),
    compiler_params=pltpu.CompilerParams(use_tc_tiling_on_sc=False),
    scratch_types=[
        pltpu.VMEM_SHARED((ACC_ROWS, HH), jnp.float32),
        pltpu.VMEM((KA, LANES), jnp.int32),
        pltpu.VMEM((KA, LANES), jnp.int32),
        pltpu.VMEM((BOUNCE, HH), jnp.float32),
        [pltpu.SemaphoreType.DMA] * 4,
    ],
)


def _k1_body(x, W_in, b_in, Wc0, p0, p1, h_o, dis_o, xs0_o, xs1_o):
    dis = lax.rsqrt(p0[...] + p1[...] + 1.0)
    h = jnp.dot(x[...], W_in[...], preferred_element_type=jnp.float32) + b_in[...]
    xs = jnp.dot(h, Wc0[...], preferred_element_type=jnp.float32) * dis
    h_o[...] = h
    dis_o[...] = dis
    xs0_o[...] = xs[:, :HH]
    xs1_o[...] = xs[:, HH:]


def _k2_body(a0, a1, xs0, xs1, h_prev, dis, bcp, Wc, h_o, xs0_o, xs1_o):
    agg = jnp.concatenate([a0[...] + xs0[...], a1[...] + xs1[...]], axis=1)
    out = agg * dis[...] + bcp[...]
    h = _gelu(out) + h_prev[...]
    xs = jnp.dot(h, Wc[...], preferred_element_type=jnp.float32) * dis[...]
    h_o[...] = h
    xs0_o[...] = xs[:, :HH]
    xs1_o[...] = xs[:, HH:]


def _k3_body(a0, a1, xs0, xs1, h_prev, dis, bcp, g, b, W1, b1, W2, b2, y_o):
    agg = jnp.concatenate([a0[...] + xs0[...], a1[...] + xs1[...]], axis=1)
    out = agg * dis[...] + bcp[...]
    h = _gelu(out) + h_prev[...]
    mu = jnp.mean(h, axis=-1, keepdims=True)
    var = jnp.mean((h - mu) ** 2, axis=-1, keepdims=True)
    hn = (h - mu) * lax.rsqrt(var + 1e-5) * g[...] + b[...]
    h1 = _gelu(jnp.dot(hn, W1[...], preferred_element_type=jnp.float32) + b1[...])
    y_o[...] = jnp.dot(h1, W2[...], preferred_element_type=jnp.float32) + b2[...]


def _row_spec(cols):
    return pl.BlockSpec((BR, cols), lambda i: (i, 0))


def _full_spec(r, c):
    return pl.BlockSpec((r, c), lambda i: (0, 0))


_k1 = pl.pallas_call(
    _k1_body,
    grid=(GRID,),
    in_specs=[
        _row_spec(128), _full_spec(128, H), _full_spec(1, H), _full_spec(H, H),
        _row_spec(1), _row_spec(1),
    ],
    out_specs=[_row_spec(H), _row_spec(1), _row_spec(HH), _row_spec(HH)],
    out_shape=[
        jax.ShapeDtypeStruct((N, H), jnp.float32),
        jax.ShapeDtypeStruct((N, 1), jnp.float32),
        jax.ShapeDtypeStruct((N, HH), jnp.float32),
        jax.ShapeDtypeStruct((N, HH), jnp.float32),
    ],
)

_k2 = pl.pallas_call(
    _k2_body,
    grid=(GRID,),
    in_specs=[
        _row_spec(HH), _row_spec(HH), _row_spec(HH), _row_spec(HH),
        _row_spec(H), _row_spec(1), _full_spec(1, H), _full_spec(H, H),
    ],
    out_specs=[_row_spec(H), _row_spec(HH), _row_spec(HH)],
    out_shape=[
        jax.ShapeDtypeStruct((N, H), jnp.float32),
        jax.ShapeDtypeStruct((N, HH), jnp.float32),
        jax.ShapeDtypeStruct((N, HH), jnp.float32),
    ],
)

_k3 = pl.pallas_call(
    _k3_body,
    grid=(GRID,),
    in_specs=[
        _row_spec(HH), _row_spec(HH), _row_spec(HH), _row_spec(HH),
        _row_spec(H), _row_spec(1), _full_spec(1, H), _full_spec(1, H),
        _full_spec(1, H), _full_spec(H, H), _full_spec(1, H), _full_spec(H, 1),
        _full_spec(1, 1),
    ],
    out_specs=[_row_spec(1)],
    out_shape=[jax.ShapeDtypeStruct((N, 1), jnp.float32)],
)


def kernel(x, edge_index, W_in, b_in, Wc, bc, ln_g, ln_b, W_h1, b_h1, W_h2, b_h2):
    ei = edge_index.astype(jnp.int32)
    pad = EPAD - E
    row = jnp.concatenate([ei[0], jnp.zeros((pad,), jnp.int32)])
    col = jnp.concatenate([ei[1], jnp.full((pad,), TRASH, jnp.int32)])
    rowh = row.reshape(EROWS, LANES)
    colh = col.reshape(EROWS, LANES)
    zeros32 = jnp.zeros((BOUNCE, HH), jnp.float32)
    zeros1 = jnp.zeros((ZROWS,), jnp.float32)
    ones128 = jnp.ones((LANES,), jnp.float32)

    p0, p1 = _deg_call(colh, zeros1, ones128)
    h, dis, xs0, xs1 = _k1(
        x, W_in, b_in.reshape(1, H), Wc[0],
        p0.reshape(ACC_ROWS, 1), p1.reshape(ACC_ROWS, 1),
    )
    for i in range(Wc.shape[0]):
        a0, a1 = _agg_call(xs0, xs1, rowh, colh, zeros32)
        if i + 1 < Wc.shape[0]:
            h, xs0, xs1 = _k2(a0, a1, xs0, xs1, h, dis,
                              bc[i].reshape(1, H), Wc[i + 1])
        else:
            (y,) = _k3(a0, a1, xs0, xs1, h, dis, bc[i].reshape(1, H),
                       ln_g.reshape(1, H), ln_b.reshape(1, H),
                       W_h1, b_h1.reshape(1, H), W_h2, b_h2.reshape(1, 1))
    return y


# speedup vs baseline: 12.6584x; 1.0427x over previous
import math

import jax
import jax.numpy as jnp
from jax import lax
from jax.experimental import pallas as pl
from jax.experimental.pallas import tpu as pltpu
from jax.experimental.pallas import tpu_sc as plsc

N = 50000
H = 64
HH = H // 2
NC, NS = 2, 16
LANES = 128
KA = 8
KR = 2
ZCH = KR * 128
E = 800000
EROWS = 6400
EPAD = EROWS * LANES
DEG_TROWS = EROWS // (NC * NS)
AGG_TROWS = EROWS // NS
ZROWS = 3128
ACC_ROWS = ZROWS * NS
TRASH = N
NZC = ZROWS // ZCH
ZTAIL = ZROWS - NZC * ZCH
BR = 2000
GRID = N // BR

_SQRT2 = math.sqrt(2.0)


def _gelu(x):
    return 0.5 * x * (1.0 + lax.erf(x / _SQRT2))


def _deg_body(colh, zeros1, ones_hbm, out0, out1, acc, cidx, ones_v, bounce):
    c = lax.axis_index("c")
    s = lax.axis_index("s")
    wid = c * NS + s
    pltpu.sync_copy(zeros1, bounce)
    pltpu.sync_copy(bounce, acc.at[pl.ds(s * ZROWS, ZROWS)])
    pltpu.sync_copy(ones_hbm, ones_v)
    plsc.subcore_barrier()

    base = wid * DEG_TROWS

    def chunk(i, carry):
        off = base + i * KA
        pltpu.sync_copy(colh.at[pl.ds(off, KA)], cidx)
        for j in range(KA):
            pltpu.sync_copy(ones_v, acc.at[cidx.at[j]], add=True)
        return carry

    lax.fori_loop(0, DEG_TROWS // KA, chunk, 0)
    plsc.subcore_barrier()

    @pl.when(c == 0)
    def _():
        pltpu.sync_copy(acc.at[pl.ds(s * ZROWS, ZROWS)], bounce)
        pltpu.sync_copy(bounce, out0.at[pl.ds(s * ZROWS, ZROWS)])

    @pl.when(c == 1)
    def _():
        pltpu.sync_copy(acc.at[pl.ds(s * ZROWS, ZROWS)], bounce)
        pltpu.sync_copy(bounce, out1.at[pl.ds(s * ZROWS, ZROWS)])


_deg_call = pl.kernel(
    _deg_body,
    out_type=[jax.ShapeDtypeStruct((ACC_ROWS,), jnp.float32)] * 2,
    mesh=plsc.VectorSubcoreMesh(
        core_axis_name="c", subcore_axis_name="s", num_cores=NC, num_subcores=NS
    ),
    scratch_types=[
        pltpu.VMEM_SHARED((ACC_ROWS,), jnp.float32),
        pltpu.VMEM((KA, LANES), jnp.int32),
        pltpu.VMEM((LANES,), jnp.float32),
        pltpu.VMEM((ZROWS,), jnp.float32),
    ],
)


def _agg_pipe(xs_hbm, out_hbm, rowh, colh, zeros32, acc,
              ridxA, cidxA, rowsA, ridxB, cidxB, rowsB, sems, s):
    gA, gB, sA, sB = sems
    a0 = s * ZROWS

    pltpu.sync_copy(zeros32, rowsA)
    zcps = [
        pltpu.async_copy(rowsA, acc.at[pl.ds(a0 + q * ZCH, ZCH)], gA)
        for q in range(NZC)
    ]
    zcps.append(pltpu.async_copy(rowsA.at[pl.ds(0, ZTAIL)],
                                 acc.at[pl.ds(a0 + NZC * ZCH, ZTAIL)], gA))
    for cp in zcps:
        cp.wait()
    plsc.subcore_barrier()

    base = s * AGG_TROWS

    def load_fire(off, ridx, cidx, rows, gsem):
        pltpu.sync_copy(rowh.at[pl.ds(off, KR)], ridx)
        pltpu.sync_copy(colh.at[pl.ds(off, KR)], cidx)
        return [
            pltpu.async_copy(xs_hbm.at[ridx.at[j]],
                             rows.at[pl.ds(j * LANES, LANES)], gsem)
            for j in range(KR)
        ]

    def fire_scatter(cidx, rows, ssem):
        return [
            pltpu.async_copy(rows.at[pl.ds(j * LANES, LANES)],
                             acc.at[cidx.at[j]], ssem, add=True)
            for j in range(KR)
        ]

    def drain_scatter(cidx, rows, ssem):
        for j in range(KR):
            pltpu.make_async_copy(rows.at[pl.ds(j * LANES, LANES)],
                                  acc.at[cidx.at[j]], ssem).wait()

    def body(i, carry):
        offA = base + i * (2 * KR)

        @pl.when(i > 0)
        def _():
            drain_scatter(cidxA, rowsA, sA)

        gac = load_fire(offA, ridxA, cidxA, rowsA, gA)

        @pl.when(i > 0)
        def _():
            drain_scatter(cidxB, rowsB, sB)

        gbc = load_fire(offA + KR, ridxB, cidxB, rowsB, gB)
        for cp in gac:
            cp.wait()
        fire_scatter(cidxA, rowsA, sA)
        for cp in gbc:
            cp.wait()
        fire_scatter(cidxB, rowsB, sB)
        return carry

    lax.fori_loop(0, AGG_TROWS // (2 * KR), body, 0)
    drain_scatter(cidxA, rowsA, sA)
    drain_scatter(cidxB, rowsB, sB)
    plsc.subcore_barrier()

    chunks = [(q * ZCH, ZCH) for q in range(NZC)] + [(NZC * ZCH, ZTAIL)]
    bufs = [rowsA, rowsB]
    pend = [None, None]
    rd = pltpu.async_copy(acc.at[pl.ds(a0 + chunks[0][0], chunks[0][1])],
                          rowsA.at[pl.ds(0, chunks[0][1])], gA)
    for q, (off, cnt) in enumerate(chunks):
        b = q % 2
        rd.wait()
        pend[b] = pltpu.async_copy(bufs[b].at[pl.ds(0, cnt)],
                                   out_hbm.at[pl.ds(a0 + off, cnt)], sA)
        if q + 1 < len(chunks):
            nb = (q + 1) % 2
            if pend[nb] is not None:
                pend[nb].wait()
                pend[nb] = None
            noff, ncnt = chunks[q + 1]
            rd = pltpu.async_copy(acc.at[pl.ds(a0 + noff, ncnt)],
                                  bufs[nb].at[pl.ds(0, ncnt)], gA)
    for cp in pend:
        if cp is not None:
            cp.wait()


def _agg_body(xs0, xs1, rowh, colh, zeros32, out0, out1, acc, ridxA, cidxA,
              rowsA, ridxB, cidxB, rowsB, sems):
    c = lax.axis_index("c")
    s = lax.axis_index("s")

    @pl.when(c == 0)
    def _():
        _agg_pipe(xs0, out0, rowh, colh, zeros32, acc,
                  ridxA, cidxA, rowsA, ridxB, cidxB, rowsB, sems, s)

    @pl.when(c == 1)
    def _():
        _agg_pipe(xs1, out1, rowh, colh, zeros32, acc,
                  ridxA, cidxA, rowsA, ridxB, cidxB, rowsB, sems, s)


_agg_call = pl.kernel(
    _agg_body,
    out_type=[jax.ShapeDtypeStruct((ACC_ROWS, HH), jnp.float32)] * 2,
    mesh=plsc.VectorSubcoreMesh(
        core_axis_name="c", subcore_axis_name="s", num_cores=NC, num_subcores=NS
    ),
    compiler_params=pltpu.CompilerParams(use_tc_tiling_on_sc=False),
    scratch_types=[
        pltpu.VMEM_SHARED((ACC_ROWS, HH), jnp.float32),
        pltpu.VMEM((KR, LANES), jnp.int32),
        pltpu.VMEM((KR, LANES), jnp.int32),
        pltpu.VMEM((ZCH, HH), jnp.float32),
        pltpu.VMEM((KR, LANES), jnp.int32),
        pltpu.VMEM((KR, LANES), jnp.int32),
        pltpu.VMEM((ZCH, HH), jnp.float32),
        [pltpu.SemaphoreType.DMA] * 4,
    ],
)


def _k1_body(x, W_in, b_in, Wc0, p0, p1, h_o, dis_o, xs0_o, xs1_o):
    dis = lax.rsqrt(p0[...] + p1[...] + 1.0)
    h = jnp.dot(x[...], W_in[...], preferred_element_type=jnp.float32) + b_in[...]
    xs = jnp.dot(h, Wc0[...], preferred_element_type=jnp.float32) * dis
    h_o[...] = h
    dis_o[...] = dis
    xs0_o[...] = xs[:, :HH]
    xs1_o[...] = xs[:, HH:]


def _k2_body(a0, a1, xs0, xs1, h_prev, dis, bcp, Wc, h_o, xs0_o, xs1_o):
    agg = jnp.concatenate([a0[...] + xs0[...], a1[...] + xs1[...]], axis=1)
    out = agg * dis[...] + bcp[...]
    h = _gelu(out) + h_prev[...]
    xs = jnp.dot(h, Wc[...], preferred_element_type=jnp.float32) * dis[...]
    h_o[...] = h
    xs0_o[...] = xs[:, :HH]
    xs1_o[...] = xs[:, HH:]


def _k3_body(a0, a1, xs0, xs1, h_prev, dis, bcp, g, b, W1, b1, W2, b2, y_o):
    agg = jnp.concatenate([a0[...] + xs0[...], a1[...] + xs1[...]], axis=1)
    out = agg * dis[...] + bcp[...]
    h = _gelu(out) + h_prev[...]
    mu = jnp.mean(h, axis=-1, keepdims=True)
    var = jnp.mean((h - mu) ** 2, axis=-1, keepdims=True)
    hn = (h - mu) * lax.rsqrt(var + 1e-5) * g[...] + b[...]
    h1 = _gelu(jnp.dot(hn, W1[...], preferred_element_type=jnp.float32) + b1[...])
    y_o[...] = jnp.dot(h1, W2[...], preferred_element_type=jnp.float32) + b2[...]


def _row_spec(cols):
    return pl.BlockSpec((BR, cols), lambda i: (i, 0))


def _full_spec(r, c):
    return pl.BlockSpec((r, c), lambda i: (0, 0))


_k1 = pl.pallas_call(
    _k1_body,
    grid=(GRID,),
    in_specs=[
        _row_spec(128), _full_spec(128, H), _full_spec(1, H), _full_spec(H, H),
        _row_spec(1), _row_spec(1),
    ],
    out_specs=[_row_spec(H), _row_spec(1), _row_spec(HH), _row_spec(HH)],
    out_shape=[
        jax.ShapeDtypeStruct((N, H), jnp.float32),
        jax.ShapeDtypeStruct((N, 1), jnp.float32),
        jax.ShapeDtypeStruct((N, HH), jnp.float32),
        jax.ShapeDtypeStruct((N, HH), jnp.float32),
    ],
)

_k2 = pl.pallas_call(
    _k2_body,
    grid=(GRID,),
    in_specs=[
        _row_spec(HH), _row_spec(HH), _row_spec(HH), _row_spec(HH),
        _row_spec(H), _row_spec(1), _full_spec(1, H), _full_spec(H, H),
    ],
    out_specs=[_row_spec(H), _row_spec(HH), _row_spec(HH)],
    out_shape=[
        jax.ShapeDtypeStruct((N, H), jnp.float32),
        jax.ShapeDtypeStruct((N, HH), jnp.float32),
        jax.ShapeDtypeStruct((N, HH), jnp.float32),
    ],
)

_k3 = pl.pallas_call(
    _k3_body,
    grid=(GRID,),
    in_specs=[
        _row_spec(HH), _row_spec(HH), _row_spec(HH), _row_spec(HH),
        _row_spec(H), _row_spec(1), _full_spec(1, H), _full_spec(1, H),
        _full_spec(1, H), _full_spec(H, H), _full_spec(1, H), _full_spec(H, 1),
        _full_spec(1, 1),
    ],
    out_specs=[_row_spec(1)],
    out_shape=[jax.ShapeDtypeStruct((N, 1), jnp.float32)],
)


def kernel(x, edge_index, W_in, b_in, Wc, bc, ln_g, ln_b, W_h1, b_h1, W_h2, b_h2):
    ei = edge_index.astype(jnp.int32)
    pad = EPAD - E
    row = jnp.concatenate([ei[0], jnp.zeros((pad,), jnp.int32)])
    col = jnp.concatenate([ei[1], jnp.full((pad,), TRASH, jnp.int32)])
    rowh = row.reshape(EROWS, LANES)
    colh = col.reshape(EROWS, LANES)
    zeros32 = jnp.zeros((ZCH, HH), jnp.float32)
    zeros1 = jnp.zeros((ZROWS,), jnp.float32)
    ones128 = jnp.ones((LANES,), jnp.float32)

    p0, p1 = _deg_call(colh, zeros1, ones128)
    h, dis, xs0, xs1 = _k1(
        x, W_in, b_in.reshape(1, H), Wc[0],
        p0.reshape(ACC_ROWS, 1), p1.reshape(ACC_ROWS, 1),
    )
    for i in range(Wc.shape[0]):
        a0, a1 = _agg_call(xs0, xs1, rowh, colh, zeros32)
        if i + 1 < Wc.shape[0]:
            h, xs0, xs1 = _k2(a0, a1, xs0, xs1, h, dis,
                              bc[i].reshape(1, H), Wc[i + 1])
        else:
            (y,) = _k3(a0, a1, xs0, xs1, h, dis, bc[i].reshape(1, H),
                       ln_g.reshape(1, H), ln_b.reshape(1, H),
                       W_h1, b_h1.reshape(1, H), W_h2, b_h2.reshape(1, 1))
    return y


# speedup vs baseline: 14.1250x; 1.1159x over previous
import math

import jax
import jax.numpy as jnp
from jax import lax
from jax.experimental import pallas as pl
from jax.experimental.pallas import tpu as pltpu
from jax.experimental.pallas import tpu_sc as plsc

N = 50000
H = 64
HH = H // 2
NC, NS = 2, 16
LANES = 128
KA = 8
KI = 10
NB = 5
LAG = 2
ZCH = 128
E = 800000
EROWS = 6400
EPAD = EROWS * LANES
DEG_TROWS = EROWS // (NC * NS)
AGG_TROWS = EROWS // NS
ZROWS = 3128
ACC_ROWS = ZROWS * NS
TRASH = N
NZC = ZROWS // ZCH
ZTAIL = ZROWS - NZC * ZCH
BR = 2000
GRID = N // BR

_SQRT2 = math.sqrt(2.0)


def _gelu(x):
    return 0.5 * x * (1.0 + lax.erf(x / _SQRT2))


def _deg_body(colh, zeros1, ones_hbm, out0, out1, acc, cidx, ones_v, bounce):
    c = lax.axis_index("c")
    s = lax.axis_index("s")
    wid = c * NS + s
    pltpu.sync_copy(zeros1, bounce)
    pltpu.sync_copy(bounce, acc.at[pl.ds(s * ZROWS, ZROWS)])
    pltpu.sync_copy(ones_hbm, ones_v)
    plsc.subcore_barrier()

    base = wid * DEG_TROWS

    def chunk(i, carry):
        off = base + i * KA
        pltpu.sync_copy(colh.at[pl.ds(off, KA)], cidx)
        for j in range(KA):
            pltpu.sync_copy(ones_v, acc.at[cidx.at[j]], add=True)
        return carry

    lax.fori_loop(0, DEG_TROWS // KA, chunk, 0)
    plsc.subcore_barrier()

    @pl.when(c == 0)
    def _():
        pltpu.sync_copy(acc.at[pl.ds(s * ZROWS, ZROWS)], bounce)
        pltpu.sync_copy(bounce, out0.at[pl.ds(s * ZROWS, ZROWS)])

    @pl.when(c == 1)
    def _():
        pltpu.sync_copy(acc.at[pl.ds(s * ZROWS, ZROWS)], bounce)
        pltpu.sync_copy(bounce, out1.at[pl.ds(s * ZROWS, ZROWS)])


_deg_call = pl.kernel(
    _deg_body,
    out_type=[jax.ShapeDtypeStruct((ACC_ROWS,), jnp.float32)] * 2,
    mesh=plsc.VectorSubcoreMesh(
        core_axis_name="c", subcore_axis_name="s", num_cores=NC, num_subcores=NS
    ),
    scratch_types=[
        pltpu.VMEM_SHARED((ACC_ROWS,), jnp.float32),
        pltpu.VMEM((KA, LANES), jnp.int32),
        pltpu.VMEM((LANES,), jnp.float32),
        pltpu.VMEM((ZROWS,), jnp.float32),
    ],
)


def _agg_pipe(xs_hbm, out_hbm, rowh, colh, zeros32, acc, ridx, cidx, rbufs,
              sems, s):
    a0 = s * ZROWS
    gsems, ssems = sems[:NB], sems[NB:]

    pltpu.sync_copy(zeros32, rbufs[0])
    zcps = [
        pltpu.async_copy(rbufs[0], acc.at[pl.ds(a0 + q * ZCH, ZCH)], gsems[0])
        for q in range(NZC)
    ]
    zcps.append(pltpu.async_copy(rbufs[0].at[pl.ds(0, ZTAIL)],
                                 acc.at[pl.ds(a0 + NZC * ZCH, ZTAIL)],
                                 gsems[0]))
    for cp in zcps:
        cp.wait()
    plsc.subcore_barrier()

    base = s * AGG_TROWS

    def fire_g(j, ridx):
        return pltpu.async_copy(xs_hbm.at[ridx.at[j]], rbufs[j % NB],
                                gsems[j % NB])

    def fire_s(j, cidx):
        return pltpu.async_copy(rbufs[j % NB], acc.at[cidx.at[j]],
                                ssems[j % NB], add=True)

    def drain_s(j, cidx):
        pltpu.make_async_copy(rbufs[j % NB], acc.at[cidx.at[j]],
                              ssems[j % NB]).wait()

    def body(i, carry):
        off = base + i * KI

        @pl.when(i > 0)
        def _():
            for j in range(KI - NB, KI):
                drain_s(j, cidx)

        pltpu.sync_copy(rowh.at[pl.ds(off, KI)], ridx)
        pltpu.sync_copy(colh.at[pl.ds(off, KI)], cidx)

        gs = {}
        for j in range(KI + LAG):
            if j < KI:
                if j >= NB:
                    drain_s(j - NB, cidx)
                gs[j] = fire_g(j, ridx)
            k = j - LAG
            if 0 <= k < KI:
                gs[k].wait()
                fire_s(k, cidx)
        return carry

    lax.fori_loop(0, AGG_TROWS // KI, body, 0)
    for j in range(KI - NB, KI):
        drain_s(j, cidx)
    plsc.subcore_barrier()

    chunks = [(q * ZCH, ZCH) for q in range(NZC)] + [(NZC * ZCH, ZTAIL)]
    bufs = [rbufs[0], rbufs[1]]
    pend = [None, None]
    rd = pltpu.async_copy(acc.at[pl.ds(a0 + chunks[0][0], chunks[0][1])],
                          bufs[0].at[pl.ds(0, chunks[0][1])], gsems[0])
    for q, (off, cnt) in enumerate(chunks):
        b = q % 2
        rd.wait()
        pend[b] = pltpu.async_copy(bufs[b].at[pl.ds(0, cnt)],
                                   out_hbm.at[pl.ds(a0 + off, cnt)], ssems[b])
        if q + 1 < len(chunks):
            nb = (q + 1) % 2
            if pend[nb] is not None:
                pend[nb].wait()
                pend[nb] = None
            noff, ncnt = chunks[q + 1]
            rd = pltpu.async_copy(acc.at[pl.ds(a0 + noff, ncnt)],
                                  bufs[nb].at[pl.ds(0, ncnt)], gsems[nb])
    for cp in pend:
        if cp is not None:
            cp.wait()


def _agg_body(xs0, xs1, rowh, colh, zeros32, out0, out1, acc, ridx, cidx,
              b0, b1, b2, b3, b4, sems):
    c = lax.axis_index("c")
    s = lax.axis_index("s")
    rbufs = [b0, b1, b2, b3, b4]

    @pl.when(c == 0)
    def _():
        _agg_pipe(xs0, out0, rowh, colh, zeros32, acc, ridx, cidx, rbufs,
                  sems, s)

    @pl.when(c == 1)
    def _():
        _agg_pipe(xs1, out1, rowh, colh, zeros32, acc, ridx, cidx, rbufs,
                  sems, s)


_agg_call = pl.kernel(
    _agg_body,
    out_type=[jax.ShapeDtypeStruct((ACC_ROWS, HH), jnp.float32)] * 2,
    mesh=plsc.VectorSubcoreMesh(
        core_axis_name="c", subcore_axis_name="s", num_cores=NC, num_subcores=NS
    ),
    compiler_params=pltpu.CompilerParams(use_tc_tiling_on_sc=False),
    scratch_types=[
        pltpu.VMEM_SHARED((ACC_ROWS, HH), jnp.float32),
        pltpu.VMEM((KI, LANES), jnp.int32),
        pltpu.VMEM((KI, LANES), jnp.int32),
        pltpu.VMEM((LANES, HH), jnp.float32),
        pltpu.VMEM((LANES, HH), jnp.float32),
        pltpu.VMEM((LANES, HH), jnp.float32),
        pltpu.VMEM((LANES, HH), jnp.float32),
        pltpu.VMEM((LANES, HH), jnp.float32),
        [pltpu.SemaphoreType.DMA] * (2 * NB),
    ],
)


def _k1_body(x, W_in, b_in, Wc0, p0, p1, h_o, dis_o, xs0_o, xs1_o):
    dis = lax.rsqrt(p0[...] + p1[...] + 1.0)
    h = jnp.dot(x[...], W_in[...], preferred_element_type=jnp.float32) + b_in[...]
    xs = jnp.dot(h, Wc0[...], preferred_element_type=jnp.float32) * dis
    h_o[...] = h
    dis_o[...] = dis
    xs0_o[...] = xs[:, :HH]
    xs1_o[...] = xs[:, HH:]


def _k2_body(a0, a1, xs0, xs1, h_prev, dis, bcp, Wc, h_o, xs0_o, xs1_o):
    agg = jnp.concatenate([a0[...] + xs0[...], a1[...] + xs1[...]], axis=1)
    out = agg * dis[...] + bcp[...]
    h = _gelu(out) + h_prev[...]
    xs = jnp.dot(h, Wc[...], preferred_element_type=jnp.float32) * dis[...]
    h_o[...] = h
    xs0_o[...] = xs[:, :HH]
    xs1_o[...] = xs[:, HH:]


def _k3_body(a0, a1, xs0, xs1, h_prev, dis, bcp, g, b, W1, b1, W2, b2, y_o):
    agg = jnp.concatenate([a0[...] + xs0[...], a1[...] + xs1[...]], axis=1)
    out = agg * dis[...] + bcp[...]
    h = _gelu(out) + h_prev[...]
    mu = jnp.mean(h, axis=-1, keepdims=True)
    var = jnp.mean((h - mu) ** 2, axis=-1, keepdims=True)
    hn = (h - mu) * lax.rsqrt(var + 1e-5) * g[...] + b[...]
    h1 = _gelu(jnp.dot(hn, W1[...], preferred_element_type=jnp.float32) + b1[...])
    y_o[...] = jnp.dot(h1, W2[...], preferred_element_type=jnp.float32) + b2[...]


def _row_spec(cols):
    return pl.BlockSpec((BR, cols), lambda i: (i, 0))


def _full_spec(r, c):
    return pl.BlockSpec((r, c), lambda i: (0, 0))


_k1 = pl.pallas_call(
    _k1_body,
    grid=(GRID,),
    in_specs=[
        _row_spec(128), _full_spec(128, H), _full_spec(1, H), _full_spec(H, H),
        _row_spec(1), _row_spec(1),
    ],
    out_specs=[_row_spec(H), _row_spec(1), _row_spec(HH), _row_spec(HH)],
    out_shape=[
        jax.ShapeDtypeStruct((N, H), jnp.float32),
        jax.ShapeDtypeStruct((N, 1), jnp.float32),
        jax.ShapeDtypeStruct((N, HH), jnp.float32),
        jax.ShapeDtypeStruct((N, HH), jnp.float32),
    ],
)

_k2 = pl.pallas_call(
    _k2_body,
    grid=(GRID,),
    in_specs=[
        _row_spec(HH), _row_spec(HH), _row_spec(HH), _row_spec(HH),
        _row_spec(H), _row_spec(1), _full_spec(1, H), _full_spec(H, H),
    ],
    out_specs=[_row_spec(H), _row_spec(HH), _row_spec(HH)],
    out_shape=[
        jax.ShapeDtypeStruct((N, H), jnp.float32),
        jax.ShapeDtypeStruct((N, HH), jnp.float32),
        jax.ShapeDtypeStruct((N, HH), jnp.float32),
    ],
)

_k3 = pl.pallas_call(
    _k3_body,
    grid=(GRID,),
    in_specs=[
        _row_spec(HH), _row_spec(HH), _row_spec(HH), _row_spec(HH),
        _row_spec(H), _row_spec(1), _full_spec(1, H), _full_spec(1, H),
        _full_spec(1, H), _full_spec(H, H), _full_spec(1, H), _full_spec(H, 1),
        _full_spec(1, 1),
    ],
    out_specs=[_row_spec(1)],
    out_shape=[jax.ShapeDtypeStruct((N, 1), jnp.float32)],
)


def kernel(x, edge_index, W_in, b_in, Wc, bc, ln_g, ln_b, W_h1, b_h1, W_h2, b_h2):
    ei = edge_index.astype(jnp.int32)
    pad = EPAD - E
    row = jnp.concatenate([ei[0], jnp.zeros((pad,), jnp.int32)])
    col = jnp.concatenate([ei[1], jnp.full((pad,), TRASH, jnp.int32)])
    rowh = row.reshape(EROWS, LANES)
    colh = col.reshape(EROWS, LANES)
    zeros32 = jnp.zeros((ZCH, HH), jnp.float32)
    zeros1 = jnp.zeros((ZROWS,), jnp.float32)
    ones128 = jnp.ones((LANES,), jnp.float32)

    p0, p1 = _deg_call(colh, zeros1, ones128)
    h, dis, xs0, xs1 = _k1(
        x, W_in, b_in.reshape(1, H), Wc[0],
        p0.reshape(ACC_ROWS, 1), p1.reshape(ACC_ROWS, 1),
    )
    for i in range(Wc.shape[0]):
        a0, a1 = _agg_call(xs0, xs1, rowh, colh, zeros32)
        if i + 1 < Wc.shape[0]:
            h, xs0, xs1 = _k2(a0, a1, xs0, xs1, h, dis,
                              bc[i].reshape(1, H), Wc[i + 1])
        else:
            (y,) = _k3(a0, a1, xs0, xs1, h, dis, bc[i].reshape(1, H),
                       ln_g.reshape(1, H), ln_b.reshape(1, H),
                       W_h1, b_h1.reshape(1, H), W_h2, b_h2.reshape(1, 1))
    return y


# speedup vs baseline: 14.8677x; 1.0526x over previous
import math

import jax
import jax.numpy as jnp
from jax import lax
from jax.experimental import pallas as pl
from jax.experimental.pallas import tpu as pltpu
from jax.experimental.pallas import tpu_sc as plsc

N = 50000
H = 64
HH = H // 2
NC, NS = 2, 16
LANES = 128
KA = 8
KI = 20
NB = 5
LAG = 2
ZCH = 128
E = 800000
EROWS = 6400
EPAD = EROWS * LANES
DEG_TROWS = EROWS // (NC * NS)
AGG_TROWS = EROWS // NS
ZROWS = 3128
ACC_ROWS = ZROWS * NS
TRASH = N
NZC = ZROWS // ZCH
ZTAIL = ZROWS - NZC * ZCH
BR = 2000
GRID = N // BR

_SQRT2 = math.sqrt(2.0)


def _gelu(x):
    return 0.5 * x * (1.0 + lax.erf(x / _SQRT2))


def _deg_body(colh, zeros1, ones_hbm, out0, out1, acc, cidx, ones_v, bounce):
    c = lax.axis_index("c")
    s = lax.axis_index("s")
    wid = c * NS + s
    pltpu.sync_copy(zeros1, bounce)
    pltpu.sync_copy(bounce, acc.at[pl.ds(s * ZROWS, ZROWS)])
    pltpu.sync_copy(ones_hbm, ones_v)
    plsc.subcore_barrier()

    base = wid * DEG_TROWS

    def chunk(i, carry):
        off = base + i * KA
        pltpu.sync_copy(colh.at[pl.ds(off, KA)], cidx)
        for j in range(KA):
            pltpu.sync_copy(ones_v, acc.at[cidx.at[j]], add=True)
        return carry

    lax.fori_loop(0, DEG_TROWS // KA, chunk, 0)
    plsc.subcore_barrier()

    @pl.when(c == 0)
    def _():
        pltpu.sync_copy(acc.at[pl.ds(s * ZROWS, ZROWS)], bounce)
        pltpu.sync_copy(bounce, out0.at[pl.ds(s * ZROWS, ZROWS)])

    @pl.when(c == 1)
    def _():
        pltpu.sync_copy(acc.at[pl.ds(s * ZROWS, ZROWS)], bounce)
        pltpu.sync_copy(bounce, out1.at[pl.ds(s * ZROWS, ZROWS)])


_deg_call = pl.kernel(
    _deg_body,
    out_type=[jax.ShapeDtypeStruct((ACC_ROWS,), jnp.float32)] * 2,
    mesh=plsc.VectorSubcoreMesh(
        core_axis_name="c", subcore_axis_name="s", num_cores=NC, num_subcores=NS
    ),
    scratch_types=[
        pltpu.VMEM_SHARED((ACC_ROWS,), jnp.float32),
        pltpu.VMEM((KA, LANES), jnp.int32),
        pltpu.VMEM((LANES,), jnp.float32),
        pltpu.VMEM((ZROWS,), jnp.float32),
    ],
)


def _agg_pipe(xs_hbm, out_hbm, rowh, colh, zeros32, acc, ridx, cidx, rbufs,
              sems, s):
    a0 = s * ZROWS
    gsems, ssems = sems[:NB], sems[NB:]

    pltpu.sync_copy(zeros32, rbufs[0])
    zcps = [
        pltpu.async_copy(rbufs[0], acc.at[pl.ds(a0 + q * ZCH, ZCH)], gsems[0])
        for q in range(NZC)
    ]
    zcps.append(pltpu.async_copy(rbufs[0].at[pl.ds(0, ZTAIL)],
                                 acc.at[pl.ds(a0 + NZC * ZCH, ZTAIL)],
                                 gsems[0]))
    for cp in zcps:
        cp.wait()
    plsc.subcore_barrier()

    base = s * AGG_TROWS

    def fire_g(j, ridx):
        return pltpu.async_copy(xs_hbm.at[ridx.at[j]], rbufs[j % NB],
                                gsems[j % NB])

    def fire_s(j, cidx):
        return pltpu.async_copy(rbufs[j % NB], acc.at[cidx.at[j]],
                                ssems[j % NB], add=True)

    def drain_s(j, cidx):
        pltpu.make_async_copy(rbufs[j % NB], acc.at[cidx.at[j]],
                              ssems[j % NB]).wait()

    def body(i, carry):
        off = base + i * KI

        @pl.when(i > 0)
        def _():
            for j in range(KI - NB, KI):
                drain_s(j, cidx)

        pltpu.sync_copy(rowh.at[pl.ds(off, KI)], ridx)
        pltpu.sync_copy(colh.at[pl.ds(off, KI)], cidx)

        gs = {}
        for j in range(KI + LAG):
            if j < KI:
                if j >= NB:
                    drain_s(j - NB, cidx)
                gs[j] = fire_g(j, ridx)
            k = j - LAG
            if 0 <= k < KI:
                gs[k].wait()
                fire_s(k, cidx)
        return carry

    lax.fori_loop(0, AGG_TROWS // KI, body, 0)
    for j in range(KI - NB, KI):
        drain_s(j, cidx)
    plsc.subcore_barrier()

    chunks = [(q * ZCH, ZCH) for q in range(NZC)] + [(NZC * ZCH, ZTAIL)]
    bufs = [rbufs[0], rbufs[1]]
    pend = [None, None]
    rd = pltpu.async_copy(acc.at[pl.ds(a0 + chunks[0][0], chunks[0][1])],
                          bufs[0].at[pl.ds(0, chunks[0][1])], gsems[0])
    for q, (off, cnt) in enumerate(chunks):
        b = q % 2
        rd.wait()
        pend[b] = pltpu.async_copy(bufs[b].at[pl.ds(0, cnt)],
                                   out_hbm.at[pl.ds(a0 + off, cnt)], ssems[b])
        if q + 1 < len(chunks):
            nb = (q + 1) % 2
            if pend[nb] is not None:
                pend[nb].wait()
                pend[nb] = None
            noff, ncnt = chunks[q + 1]
            rd = pltpu.async_copy(acc.at[pl.ds(a0 + noff, ncnt)],
                                  bufs[nb].at[pl.ds(0, ncnt)], gsems[nb])
    for cp in pend:
        if cp is not None:
            cp.wait()


def _agg_body(xs0, xs1, rowh, colh, zeros32, out0, out1, acc, ridx, cidx,
              b0, b1, b2, b3, b4, sems):
    c = lax.axis_index("c")
    s = lax.axis_index("s")
    rbufs = [b0, b1, b2, b3, b4]

    @pl.when(c == 0)
    def _():
        _agg_pipe(xs0, out0, rowh, colh, zeros32, acc, ridx, cidx, rbufs,
                  sems, s)

    @pl.when(c == 1)
    def _():
        _agg_pipe(xs1, out1, rowh, colh, zeros32, acc, ridx, cidx, rbufs,
                  sems, s)


_agg_call = pl.kernel(
    _agg_body,
    out_type=[jax.ShapeDtypeStruct((ACC_ROWS, HH), jnp.float32)] * 2,
    mesh=plsc.VectorSubcoreMesh(
        core_axis_name="c", subcore_axis_name="s", num_cores=NC, num_subcores=NS
    ),
    compiler_params=pltpu.CompilerParams(use_tc_tiling_on_sc=False),
    scratch_types=[
        pltpu.VMEM_SHARED((ACC_ROWS, HH), jnp.float32),
        pltpu.VMEM((KI, LANES), jnp.int32),
        pltpu.VMEM((KI, LANES), jnp.int32),
        pltpu.VMEM((LANES, HH), jnp.float32),
        pltpu.VMEM((LANES, HH), jnp.float32),
        pltpu.VMEM((LANES, HH), jnp.float32),
        pltpu.VMEM((LANES, HH), jnp.float32),
        pltpu.VMEM((LANES, HH), jnp.float32),
        [pltpu.SemaphoreType.DMA] * (2 * NB),
    ],
)


def _k1_body(x, W_in, b_in, Wc0, p0, p1, h_o, dis_o, xs0_o, xs1_o):
    dis = lax.rsqrt(p0[...] + p1[...] + 1.0)
    h = jnp.dot(x[...], W_in[...], preferred_element_type=jnp.float32) + b_in[...]
    xs = jnp.dot(h, Wc0[...], preferred_element_type=jnp.float32) * dis
    h_o[...] = h
    dis_o[...] = dis
    xs0_o[...] = xs[:, :HH]
    xs1_o[...] = xs[:, HH:]


def _k2_body(a0, a1, xs0, xs1, h_prev, dis, bcp, Wc, h_o, xs0_o, xs1_o):
    agg = jnp.concatenate([a0[...] + xs0[...], a1[...] + xs1[...]], axis=1)
    out = agg * dis[...] + bcp[...]
    h = _gelu(out) + h_prev[...]
    xs = jnp.dot(h, Wc[...], preferred_element_type=jnp.float32) * dis[...]
    h_o[...] = h
    xs0_o[...] = xs[:, :HH]
    xs1_o[...] = xs[:, HH:]


def _k3_body(a0, a1, xs0, xs1, h_prev, dis, bcp, g, b, W1, b1, W2, b2, y_o):
    agg = jnp.concatenate([a0[...] + xs0[...], a1[...] + xs1[...]], axis=1)
    out = agg * dis[...] + bcp[...]
    h = _gelu(out) + h_prev[...]
    mu = jnp.mean(h, axis=-1, keepdims=True)
    var = jnp.mean((h - mu) ** 2, axis=-1, keepdims=True)
    hn = (h - mu) * lax.rsqrt(var + 1e-5) * g[...] + b[...]
    h1 = _gelu(jnp.dot(hn, W1[...], preferred_element_type=jnp.float32) + b1[...])
    y_o[...] = jnp.dot(h1, W2[...], preferred_element_type=jnp.float32) + b2[...]


def _row_spec(cols):
    return pl.BlockSpec((BR, cols), lambda i: (i, 0))


def _full_spec(r, c):
    return pl.BlockSpec((r, c), lambda i: (0, 0))


_k1 = pl.pallas_call(
    _k1_body,
    grid=(GRID,),
    in_specs=[
        _row_spec(128), _full_spec(128, H), _full_spec(1, H), _full_spec(H, H),
        _row_spec(1), _row_spec(1),
    ],
    out_specs=[_row_spec(H), _row_spec(1), _row_spec(HH), _row_spec(HH)],
    out_shape=[
        jax.ShapeDtypeStruct((N, H), jnp.float32),
        jax.ShapeDtypeStruct((N, 1), jnp.float32),
        jax.ShapeDtypeStruct((N, HH), jnp.float32),
        jax.ShapeDtypeStruct((N, HH), jnp.float32),
    ],
)

_k2 = pl.pallas_call(
    _k2_body,
    grid=(GRID,),
    in_specs=[
        _row_spec(HH), _row_spec(HH), _row_spec(HH), _row_spec(HH),
        _row_spec(H), _row_spec(1), _full_spec(1, H), _full_spec(H, H),
    ],
    out_specs=[_row_spec(H), _row_spec(HH), _row_spec(HH)],
    out_shape=[
        jax.ShapeDtypeStruct((N, H), jnp.float32),
        jax.ShapeDtypeStruct((N, HH), jnp.float32),
        jax.ShapeDtypeStruct((N, HH), jnp.float32),
    ],
)

_k3 = pl.pallas_call(
    _k3_body,
    grid=(GRID,),
    in_specs=[
        _row_spec(HH), _row_spec(HH), _row_spec(HH), _row_spec(HH),
        _row_spec(H), _row_spec(1), _full_spec(1, H), _full_spec(1, H),
        _full_spec(1, H), _full_spec(H, H), _full_spec(1, H), _full_spec(H, 1),
        _full_spec(1, 1),
    ],
    out_specs=[_row_spec(1)],
    out_shape=[jax.ShapeDtypeStruct((N, 1), jnp.float32)],
)


def kernel(x, edge_index, W_in, b_in, Wc, bc, ln_g, ln_b, W_h1, b_h1, W_h2, b_h2):
    ei = edge_index.astype(jnp.int32)
    pad = EPAD - E
    row = jnp.concatenate([ei[0], jnp.zeros((pad,), jnp.int32)])
    col = jnp.concatenate([ei[1], jnp.full((pad,), TRASH, jnp.int32)])
    rowh = row.reshape(EROWS, LANES)
    colh = col.reshape(EROWS, LANES)
    zeros32 = jnp.zeros((ZCH, HH), jnp.float32)
    zeros1 = jnp.zeros((ZROWS,), jnp.float32)
    ones128 = jnp.ones((LANES,), jnp.float32)

    p0, p1 = _deg_call(colh, zeros1, ones128)
    h, dis, xs0, xs1 = _k1(
        x, W_in, b_in.reshape(1, H), Wc[0],
        p0.reshape(ACC_ROWS, 1), p1.reshape(ACC_ROWS, 1),
    )
    for i in range(Wc.shape[0]):
        a0, a1 = _agg_call(xs0, xs1, rowh, colh, zeros32)
        if i + 1 < Wc.shape[0]:
            h, xs0, xs1 = _k2(a0, a1, xs0, xs1, h, dis,
                              bc[i].reshape(1, H), Wc[i + 1])
        else:
            (y,) = _k3(a0, a1, xs0, xs1, h, dis, bc[i].reshape(1, H),
                       ln_g.reshape(1, H), ln_b.reshape(1, H),
                       W_h1, b_h1.reshape(1, H), W_h2, b_h2.reshape(1, 1))
    return y


# speedup vs baseline: 15.0488x; 1.0122x over previous
import math

import jax
import jax.numpy as jnp
from jax import lax
from jax.experimental import pallas as pl
from jax.experimental.pallas import tpu as pltpu
from jax.experimental.pallas import tpu_sc as plsc

N = 50000
H = 64
HH = H // 2
NC, NS = 2, 16
LANES = 128
KA = 8
KI = 20
NB = 5
LAG = 3
ZCH = 128
E = 800000
EROWS = 6400
EPAD = EROWS * LANES
DEG_TROWS = EROWS // (NC * NS)
AGG_TROWS = EROWS // NS
ZROWS = 3128
ACC_ROWS = ZROWS * NS
TRASH = N
NZC = ZROWS // ZCH
ZTAIL = ZROWS - NZC * ZCH
BR = 2000
GRID = N // BR

_SQRT2 = math.sqrt(2.0)


def _gelu(x):
    return 0.5 * x * (1.0 + lax.erf(x / _SQRT2))


def _deg_body(colh, zeros1, ones_hbm, out0, out1, acc, cidx, ones_v, bounce):
    c = lax.axis_index("c")
    s = lax.axis_index("s")
    wid = c * NS + s
    pltpu.sync_copy(zeros1, bounce)
    pltpu.sync_copy(bounce, acc.at[pl.ds(s * ZROWS, ZROWS)])
    pltpu.sync_copy(ones_hbm, ones_v)
    plsc.subcore_barrier()

    base = wid * DEG_TROWS

    def chunk(i, carry):
        off = base + i * KA
        pltpu.sync_copy(colh.at[pl.ds(off, KA)], cidx)
        for j in range(KA):
            pltpu.sync_copy(ones_v, acc.at[cidx.at[j]], add=True)
        return carry

    lax.fori_loop(0, DEG_TROWS // KA, chunk, 0)
    plsc.subcore_barrier()

    @pl.when(c == 0)
    def _():
        pltpu.sync_copy(acc.at[pl.ds(s * ZROWS, ZROWS)], bounce)
        pltpu.sync_copy(bounce, out0.at[pl.ds(s * ZROWS, ZROWS)])

    @pl.when(c == 1)
    def _():
        pltpu.sync_copy(acc.at[pl.ds(s * ZROWS, ZROWS)], bounce)
        pltpu.sync_copy(bounce, out1.at[pl.ds(s * ZROWS, ZROWS)])


_deg_call = pl.kernel(
    _deg_body,
    out_type=[jax.ShapeDtypeStruct((ACC_ROWS,), jnp.float32)] * 2,
    mesh=plsc.VectorSubcoreMesh(
        core_axis_name="c", subcore_axis_name="s", num_cores=NC, num_subcores=NS
    ),
    scratch_types=[
        pltpu.VMEM_SHARED((ACC_ROWS,), jnp.float32),
        pltpu.VMEM((KA, LANES), jnp.int32),
        pltpu.VMEM((LANES,), jnp.float32),
        pltpu.VMEM((ZROWS,), jnp.float32),
    ],
)


def _agg_pipe(xs_hbm, out_hbm, rowh, colh, zeros32, acc, ridx, cidx, rbufs,
              sems, s):
    a0 = s * ZROWS
    gsems, ssems = sems[:NB], sems[NB:]

    pltpu.sync_copy(zeros32, rbufs[0])
    zcps = [
        pltpu.async_copy(rbufs[0], acc.at[pl.ds(a0 + q * ZCH, ZCH)], gsems[0])
        for q in range(NZC)
    ]
    zcps.append(pltpu.async_copy(rbufs[0].at[pl.ds(0, ZTAIL)],
                                 acc.at[pl.ds(a0 + NZC * ZCH, ZTAIL)],
                                 gsems[0]))
    for cp in zcps:
        cp.wait()
    plsc.subcore_barrier()

    base = s * AGG_TROWS

    def fire_g(j, ridx):
        return pltpu.async_copy(xs_hbm.at[ridx.at[j]], rbufs[j % NB],
                                gsems[j % NB])

    def fire_s(j, cidx):
        return pltpu.async_copy(rbufs[j % NB], acc.at[cidx.at[j]],
                                ssems[j % NB], add=True)

    def drain_s(j, cidx):
        pltpu.make_async_copy(rbufs[j % NB], acc.at[cidx.at[j]],
                              ssems[j % NB]).wait()

    def body(i, carry):
        off = base + i * KI

        @pl.when(i > 0)
        def _():
            for j in range(KI - NB, KI):
                drain_s(j, cidx)

        pltpu.sync_copy(rowh.at[pl.ds(off, KI)], ridx)
        pltpu.sync_copy(colh.at[pl.ds(off, KI)], cidx)

        gs = {}
        for j in range(KI + LAG):
            if j < KI:
                if j >= NB:
                    drain_s(j - NB, cidx)
                gs[j] = fire_g(j, ridx)
            k = j - LAG
            if 0 <= k < KI:
                gs[k].wait()
                fire_s(k, cidx)
        return carry

    lax.fori_loop(0, AGG_TROWS // KI, body, 0)
    for j in range(KI - NB, KI):
        drain_s(j, cidx)
    plsc.subcore_barrier()

    chunks = [(q * ZCH, ZCH) for q in range(NZC)] + [(NZC * ZCH, ZTAIL)]
    bufs = [rbufs[0], rbufs[1]]
    pend = [None, None]
    rd = pltpu.async_copy(acc.at[pl.ds(a0 + chunks[0][0], chunks[0][1])],
                          bufs[0].at[pl.ds(0, chunks[0][1])], gsems[0])
    for q, (off, cnt) in enumerate(chunks):
        b = q % 2
        rd.wait()
        pend[b] = pltpu.async_copy(bufs[b].at[pl.ds(0, cnt)],
                                   out_hbm.at[pl.ds(a0 + off, cnt)], ssems[b])
        if q + 1 < len(chunks):
            nb = (q + 1) % 2
            if pend[nb] is not None:
                pend[nb].wait()
                pend[nb] = None
            noff, ncnt = chunks[q + 1]
            rd = pltpu.async_copy(acc.at[pl.ds(a0 + noff, ncnt)],
                                  bufs[nb].at[pl.ds(0, ncnt)], gsems[nb])
    for cp in pend:
        if cp is not None:
            cp.wait()


def _agg_body(xs0, xs1, rowh, colh, zeros32, out0, out1, acc, ridx, cidx,
              b0, b1, b2, b3, b4, sems):
    c = lax.axis_index("c")
    s = lax.axis_index("s")
    rbufs = [b0, b1, b2, b3, b4]

    @pl.when(c == 0)
    def _():
        _agg_pipe(xs0, out0, rowh, colh, zeros32, acc, ridx, cidx, rbufs,
                  sems, s)

    @pl.when(c == 1)
    def _():
        _agg_pipe(xs1, out1, rowh, colh, zeros32, acc, ridx, cidx, rbufs,
                  sems, s)


_agg_call = pl.kernel(
    _agg_body,
    out_type=[jax.ShapeDtypeStruct((ACC_ROWS, HH), jnp.float32)] * 2,
    mesh=plsc.VectorSubcoreMesh(
        core_axis_name="c", subcore_axis_name="s", num_cores=NC, num_subcores=NS
    ),
    compiler_params=pltpu.CompilerParams(use_tc_tiling_on_sc=False),
    scratch_types=[
        pltpu.VMEM_SHARED((ACC_ROWS, HH), jnp.float32),
        pltpu.VMEM((KI, LANES), jnp.int32),
        pltpu.VMEM((KI, LANES), jnp.int32),
        pltpu.VMEM((LANES, HH), jnp.float32),
        pltpu.VMEM((LANES, HH), jnp.float32),
        pltpu.VMEM((LANES, HH), jnp.float32),
        pltpu.VMEM((LANES, HH), jnp.float32),
        pltpu.VMEM((LANES, HH), jnp.float32),
        [pltpu.SemaphoreType.DMA] * (2 * NB),
    ],
)


def _k1_body(x, W_in, b_in, Wc0, p0, p1, h_o, dis_o, xs0_o, xs1_o):
    dis = lax.rsqrt(p0[...] + p1[...] + 1.0)
    h = jnp.dot(x[...], W_in[...], preferred_element_type=jnp.float32) + b_in[...]
    xs = jnp.dot(h, Wc0[...], preferred_element_type=jnp.float32) * dis
    h_o[...] = h
    dis_o[...] = dis
    xs0_o[...] = xs[:, :HH]
    xs1_o[...] = xs[:, HH:]


def _k2_body(a0, a1, xs0, xs1, h_prev, dis, bcp, Wc, h_o, xs0_o, xs1_o):
    agg = jnp.concatenate([a0[...] + xs0[...], a1[...] + xs1[...]], axis=1)
    out = agg * dis[...] + bcp[...]
    h = _gelu(out) + h_prev[...]
    xs = jnp.dot(h, Wc[...], preferred_element_type=jnp.float32) * dis[...]
    h_o[...] = h
    xs0_o[...] = xs[:, :HH]
    xs1_o[...] = xs[:, HH:]


def _k3_body(a0, a1, xs0, xs1, h_prev, dis, bcp, g, b, W1, b1, W2, b2, y_o):
    agg = jnp.concatenate([a0[...] + xs0[...], a1[...] + xs1[...]], axis=1)
    out = agg * dis[...] + bcp[...]
    h = _gelu(out) + h_prev[...]
    mu = jnp.mean(h, axis=-1, keepdims=True)
    var = jnp.mean((h - mu) ** 2, axis=-1, keepdims=True)
    hn = (h - mu) * lax.rsqrt(var + 1e-5) * g[...] + b[...]
    h1 = _gelu(jnp.dot(hn, W1[...], preferred_element_type=jnp.float32) + b1[...])
    y_o[...] = jnp.dot(h1, W2[...], preferred_element_type=jnp.float32) + b2[...]


def _row_spec(cols):
    return pl.BlockSpec((BR, cols), lambda i: (i, 0))


def _full_spec(r, c):
    return pl.BlockSpec((r, c), lambda i: (0, 0))


_k1 = pl.pallas_call(
    _k1_body,
    grid=(GRID,),
    in_specs=[
        _row_spec(128), _full_spec(128, H), _full_spec(1, H), _full_spec(H, H),
        _row_spec(1), _row_spec(1),
    ],
    out_specs=[_row_spec(H), _row_spec(1), _row_spec(HH), _row_spec(HH)],
    out_shape=[
        jax.ShapeDtypeStruct((N, H), jnp.float32),
        jax.ShapeDtypeStruct((N, 1), jnp.float32),
        jax.ShapeDtypeStruct((N, HH), jnp.float32),
        jax.ShapeDtypeStruct((N, HH), jnp.float32),
    ],
)

_k2 = pl.pallas_call(
    _k2_body,
    grid=(GRID,),
    in_specs=[
        _row_spec(HH), _row_spec(HH), _row_spec(HH), _row_spec(HH),
        _row_spec(H), _row_spec(1), _full_spec(1, H), _full_spec(H, H),
    ],
    out_specs=[_row_spec(H), _row_spec(HH), _row_spec(HH)],
    out_shape=[
        jax.ShapeDtypeStruct((N, H), jnp.float32),
        jax.ShapeDtypeStruct((N, HH), jnp.float32),
        jax.ShapeDtypeStruct((N, HH), jnp.float32),
    ],
)

_k3 = pl.pallas_call(
    _k3_body,
    grid=(GRID,),
    in_specs=[
        _row_spec(HH), _row_spec(HH), _row_spec(HH), _row_spec(HH),
        _row_spec(H), _row_spec(1), _full_spec(1, H), _full_spec(1, H),
        _full_spec(1, H), _full_spec(H, H), _full_spec(1, H), _full_spec(H, 1),
        _full_spec(1, 1),
    ],
    out_specs=[_row_spec(1)],
    out_shape=[jax.ShapeDtypeStruct((N, 1), jnp.float32)],
)


def kernel(x, edge_index, W_in, b_in, Wc, bc, ln_g, ln_b, W_h1, b_h1, W_h2, b_h2):
    ei = edge_index.astype(jnp.int32)
    pad = EPAD - E
    row = jnp.concatenate([ei[0], jnp.zeros((pad,), jnp.int32)])
    col = jnp.concatenate([ei[1], jnp.full((pad,), TRASH, jnp.int32)])
    rowh = row.reshape(EROWS, LANES)
    colh = col.reshape(EROWS, LANES)
    zeros32 = jnp.zeros((ZCH, HH), jnp.float32)
    zeros1 = jnp.zeros((ZROWS,), jnp.float32)
    ones128 = jnp.ones((LANES,), jnp.float32)

    p0, p1 = _deg_call(colh, zeros1, ones128)
    h, dis, xs0, xs1 = _k1(
        x, W_in, b_in.reshape(1, H), Wc[0],
        p0.reshape(ACC_ROWS, 1), p1.reshape(ACC_ROWS, 1),
    )
    for i in range(Wc.shape[0]):
        a0, a1 = _agg_call(xs0, xs1, rowh, colh, zeros32)
        if i + 1 < Wc.shape[0]:
            h, xs0, xs1 = _k2(a0, a1, xs0, xs1, h, dis,
                              bc[i].reshape(1, H), Wc[i + 1])
        else:
            (y,) = _k3(a0, a1, xs0, xs1, h, dis, bc[i].reshape(1, H),
                       ln_g.reshape(1, H), ln_b.reshape(1, H),
                       W_h1, b_h1.reshape(1, H), W_h2, b_h2.reshape(1, 1))
    return y


# speedup vs baseline: 15.1202x; 1.0047x over previous
import math

import jax
import jax.numpy as jnp
from jax import lax
from jax.experimental import pallas as pl
from jax.experimental.pallas import tpu as pltpu
from jax.experimental.pallas import tpu_sc as plsc

N = 50000
H = 64
HH = H // 2
NC, NS = 2, 16
LANES = 128
KA = 8
KI = 20
NB = 5
LAG = 4
ZCH = 128
_PROBE_NO_SCATTER = False
E = 800000
EROWS = 6400
EPAD = EROWS * LANES
DEG_TROWS = EROWS // (NC * NS)
AGG_TROWS = EROWS // NS
ZROWS = 3128
ACC_ROWS = ZROWS * NS
TRASH = N
NZC = ZROWS // ZCH
ZTAIL = ZROWS - NZC * ZCH
BR = 2000
GRID = N // BR

_SQRT2 = math.sqrt(2.0)


def _gelu(x):
    return 0.5 * x * (1.0 + lax.erf(x / _SQRT2))


def _deg_body(colh, zeros1, ones_hbm, out0, out1, acc, cidx, ones_v, bounce):
    c = lax.axis_index("c")
    s = lax.axis_index("s")
    wid = c * NS + s
    pltpu.sync_copy(zeros1, bounce)
    pltpu.sync_copy(bounce, acc.at[pl.ds(s * ZROWS, ZROWS)])
    pltpu.sync_copy(ones_hbm, ones_v)
    plsc.subcore_barrier()

    base = wid * DEG_TROWS

    def chunk(i, carry):
        off = base + i * KA
        pltpu.sync_copy(colh.at[pl.ds(off, KA)], cidx)
        for j in range(KA):
            pltpu.sync_copy(ones_v, acc.at[cidx.at[j]], add=True)
        return carry

    lax.fori_loop(0, DEG_TROWS // KA, chunk, 0)
    plsc.subcore_barrier()

    @pl.when(c == 0)
    def _():
        pltpu.sync_copy(acc.at[pl.ds(s * ZROWS, ZROWS)], bounce)
        pltpu.sync_copy(bounce, out0.at[pl.ds(s * ZROWS, ZROWS)])

    @pl.when(c == 1)
    def _():
        pltpu.sync_copy(acc.at[pl.ds(s * ZROWS, ZROWS)], bounce)
        pltpu.sync_copy(bounce, out1.at[pl.ds(s * ZROWS, ZROWS)])


_deg_call = pl.kernel(
    _deg_body,
    out_type=[jax.ShapeDtypeStruct((ACC_ROWS,), jnp.float32)] * 2,
    mesh=plsc.VectorSubcoreMesh(
        core_axis_name="c", subcore_axis_name="s", num_cores=NC, num_subcores=NS
    ),
    scratch_types=[
        pltpu.VMEM_SHARED((ACC_ROWS,), jnp.float32),
        pltpu.VMEM((KA, LANES), jnp.int32),
        pltpu.VMEM((LANES,), jnp.float32),
        pltpu.VMEM((ZROWS,), jnp.float32),
    ],
)


def _agg_pipe(xs_hbm, out_hbm, rowh, colh, zeros32, acc, ridx, cidx, rbufs,
              sems, s):
    a0 = s * ZROWS
    gsems, ssems = sems[:NB], sems[NB:]

    pltpu.sync_copy(zeros32, rbufs[0])
    zcps = [
        pltpu.async_copy(rbufs[0], acc.at[pl.ds(a0 + q * ZCH, ZCH)], gsems[0])
        for q in range(NZC)
    ]
    zcps.append(pltpu.async_copy(rbufs[0].at[pl.ds(0, ZTAIL)],
                                 acc.at[pl.ds(a0 + NZC * ZCH, ZTAIL)],
                                 gsems[0]))
    for cp in zcps:
        cp.wait()
    plsc.subcore_barrier()

    base = s * AGG_TROWS

    def fire_g(j, ridx):
        return pltpu.async_copy(xs_hbm.at[ridx.at[j]], rbufs[j % NB],
                                gsems[j % NB])

    def fire_s(j, cidx):
        return pltpu.async_copy(rbufs[j % NB], acc.at[cidx.at[j]],
                                ssems[j % NB], add=True)

    def drain_s(j, cidx):
        pltpu.make_async_copy(rbufs[j % NB], acc.at[cidx.at[j]],
                              ssems[j % NB]).wait()

    def body(i, carry):
        off = base + i * KI

        @pl.when(i > 0)
        def _():
            if not _PROBE_NO_SCATTER:
                for j in range(KI - NB, KI):
                    drain_s(j, cidx)

        pltpu.sync_copy(rowh.at[pl.ds(off, KI)], ridx)
        pltpu.sync_copy(colh.at[pl.ds(off, KI)], cidx)

        gs = {}
        for j in range(KI + LAG):
            if j < KI:
                if j >= NB and not _PROBE_NO_SCATTER:
                    drain_s(j - NB, cidx)
                gs[j] = fire_g(j, ridx)
            k = j - LAG
            if 0 <= k < KI:
                gs[k].wait()
                if not _PROBE_NO_SCATTER:
                    fire_s(k, cidx)
        return carry

    lax.fori_loop(0, AGG_TROWS // KI, body, 0)
    if not _PROBE_NO_SCATTER:
        for j in range(KI - NB, KI):
            drain_s(j, cidx)
    plsc.subcore_barrier()

    chunks = [(q * ZCH, ZCH) for q in range(NZC)] + [(NZC * ZCH, ZTAIL)]
    bufs = [rbufs[0], rbufs[1]]
    pend = [None, None]
    rd = pltpu.async_copy(acc.at[pl.ds(a0 + chunks[0][0], chunks[0][1])],
                          bufs[0].at[pl.ds(0, chunks[0][1])], gsems[0])
    for q, (off, cnt) in enumerate(chunks):
        b = q % 2
        rd.wait()
        pend[b] = pltpu.async_copy(bufs[b].at[pl.ds(0, cnt)],
                                   out_hbm.at[pl.ds(a0 + off, cnt)], ssems[b])
        if q + 1 < len(chunks):
            nb = (q + 1) % 2
            if pend[nb] is not None:
                pend[nb].wait()
                pend[nb] = None
            noff, ncnt = chunks[q + 1]
            rd = pltpu.async_copy(acc.at[pl.ds(a0 + noff, ncnt)],
                                  bufs[nb].at[pl.ds(0, ncnt)], gsems[nb])
    for cp in pend:
        if cp is not None:
            cp.wait()


def _agg_body(xs0, xs1, rowh, colh, zeros32, out0, out1, acc, ridx, cidx,
              b0, b1, b2, b3, b4, sems):
    c = lax.axis_index("c")
    s = lax.axis_index("s")
    rbufs = [b0, b1, b2, b3, b4]

    @pl.when(c == 0)
    def _():
        _agg_pipe(xs0, out0, rowh, colh, zeros32, acc, ridx, cidx, rbufs,
                  sems, s)

    @pl.when(c == 1)
    def _():
        _agg_pipe(xs1, out1, rowh, colh, zeros32, acc, ridx, cidx, rbufs,
                  sems, s)


_agg_call = pl.kernel(
    _agg_body,
    out_type=[jax.ShapeDtypeStruct((ACC_ROWS, HH), jnp.float32)] * 2,
    mesh=plsc.VectorSubcoreMesh(
        core_axis_name="c", subcore_axis_name="s", num_cores=NC, num_subcores=NS
    ),
    compiler_params=pltpu.CompilerParams(use_tc_tiling_on_sc=False),
    scratch_types=[
        pltpu.VMEM_SHARED((ACC_ROWS, HH), jnp.float32),
        pltpu.VMEM((KI, LANES), jnp.int32),
        pltpu.VMEM((KI, LANES), jnp.int32),
        pltpu.VMEM((LANES, HH), jnp.float32),
        pltpu.VMEM((LANES, HH), jnp.float32),
        pltpu.VMEM((LANES, HH), jnp.float32),
        pltpu.VMEM((LANES, HH), jnp.float32),
        pltpu.VMEM((LANES, HH), jnp.float32),
        [pltpu.SemaphoreType.DMA] * (2 * NB),
    ],
)


def _k1_body(x, W_in, b_in, Wc0, p0, p1, h_o, dis_o, xs0_o, xs1_o):
    dis = lax.rsqrt(p0[...] + p1[...] + 1.0)
    h = jnp.dot(x[...], W_in[...], preferred_element_type=jnp.float32) + b_in[...]
    xs = jnp.dot(h, Wc0[...], preferred_element_type=jnp.float32) * dis
    h_o[...] = h
    dis_o[...] = dis
    xs0_o[...] = xs[:, :HH]
    xs1_o[...] = xs[:, HH:]


def _k2_body(a0, a1, xs0, xs1, h_prev, dis, bcp, Wc, h_o, xs0_o, xs1_o):
    agg = jnp.concatenate([a0[...] + xs0[...], a1[...] + xs1[...]], axis=1)
    out = agg * dis[...] + bcp[...]
    h = _gelu(out) + h_prev[...]
    xs = jnp.dot(h, Wc[...], preferred_element_type=jnp.float32) * dis[...]
    h_o[...] = h
    xs0_o[...] = xs[:, :HH]
    xs1_o[...] = xs[:, HH:]


def _k3_body(a0, a1, xs0, xs1, h_prev, dis, bcp, g, b, W1, b1, W2, b2, y_o):
    agg = jnp.concatenate([a0[...] + xs0[...], a1[...] + xs1[...]], axis=1)
    out = agg * dis[...] + bcp[...]
    h = _gelu(out) + h_prev[...]
    mu = jnp.mean(h, axis=-1, keepdims=True)
    var = jnp.mean((h - mu) ** 2, axis=-1, keepdims=True)
    hn = (h - mu) * lax.rsqrt(var + 1e-5) * g[...] + b[...]
    h1 = _gelu(jnp.dot(hn, W1[...], preferred_element_type=jnp.float32) + b1[...])
    y_o[...] = jnp.dot(h1, W2[...], preferred_element_type=jnp.float32) + b2[...]


def _row_spec(cols):
    return pl.BlockSpec((BR, cols), lambda i: (i, 0))


def _full_spec(r, c):
    return pl.BlockSpec((r, c), lambda i: (0, 0))


_k1 = pl.pallas_call(
    _k1_body,
    grid=(GRID,),
    in_specs=[
        _row_spec(128), _full_spec(128, H), _full_spec(1, H), _full_spec(H, H),
        _row_spec(1), _row_spec(1),
    ],
    out_specs=[_row_spec(H), _row_spec(1), _row_spec(HH), _row_spec(HH)],
    out_shape=[
        jax.ShapeDtypeStruct((N, H), jnp.float32),
        jax.ShapeDtypeStruct((N, 1), jnp.float32),
        jax.ShapeDtypeStruct((N, HH), jnp.float32),
        jax.ShapeDtypeStruct((N, HH), jnp.float32),
    ],
)

_k2 = pl.pallas_call(
    _k2_body,
    grid=(GRID,),
    in_specs=[
        _row_spec(HH), _row_spec(HH), _row_spec(HH), _row_spec(HH),
        _row_spec(H), _row_spec(1), _full_spec(1, H), _full_spec(H, H),
    ],
    out_specs=[_row_spec(H), _row_spec(HH), _row_spec(HH)],
    out_shape=[
        jax.ShapeDtypeStruct((N, H), jnp.float32),
        jax.ShapeDtypeStruct((N, HH), jnp.float32),
        jax.ShapeDtypeStruct((N, HH), jnp.float32),
    ],
)

_k3 = pl.pallas_call(
    _k3_body,
    grid=(GRID,),
    in_specs=[
        _row_spec(HH), _row_spec(HH), _row_spec(HH), _row_spec(HH),
        _row_spec(H), _row_spec(1), _full_spec(1, H), _full_spec(1, H),
        _full_spec(1, H), _full_spec(H, H), _full_spec(1, H), _full_spec(H, 1),
        _full_spec(1, 1),
    ],
    out_specs=[_row_spec(1)],
    out_shape=[jax.ShapeDtypeStruct((N, 1), jnp.float32)],
)


def kernel(x, edge_index, W_in, b_in, Wc, bc, ln_g, ln_b, W_h1, b_h1, W_h2, b_h2):
    ei = edge_index.astype(jnp.int32)
    pad = EPAD - E
    row = jnp.concatenate([ei[0], jnp.zeros((pad,), jnp.int32)])
    col = jnp.concatenate([ei[1], jnp.full((pad,), TRASH, jnp.int32)])
    rowh = row.reshape(EROWS, LANES)
    colh = col.reshape(EROWS, LANES)
    zeros32 = jnp.zeros((ZCH, HH), jnp.float32)
    zeros1 = jnp.zeros((ZROWS,), jnp.float32)
    ones128 = jnp.ones((LANES,), jnp.float32)

    p0, p1 = _deg_call(colh, zeros1, ones128)
    h, dis, xs0, xs1 = _k1(
        x, W_in, b_in.reshape(1, H), Wc[0],
        p0.reshape(ACC_ROWS, 1), p1.reshape(ACC_ROWS, 1),
    )
    for i in range(Wc.shape[0]):
        a0, a1 = _agg_call(xs0, xs1, rowh, colh, zeros32)
        if i + 1 < Wc.shape[0]:
            h, xs0, xs1 = _k2(a0, a1, xs0, xs1, h, dis,
                              bc[i].reshape(1, H), Wc[i + 1])
        else:
            (y,) = _k3(a0, a1, xs0, xs1, h, dis, bc[i].reshape(1, H),
                       ln_g.reshape(1, H), ln_b.reshape(1, H),
                       W_h1, b_h1.reshape(1, H), W_h2, b_h2.reshape(1, 1))
    return y
